# baseline (speedup 1.0000x reference)
import jax
import jax.numpy as jnp
from jax.experimental import pallas as pl

N = 50000
D = 128
H = 4
DH = D // H
L = 2
TYPES = ['email', 'noun']
RELS = [('email', 'noun', 'e2n'), ('noun', 'email', 'n2e')]


def _segment_softmax(alpha, seg, n):
    amax = jax.ops.segment_max(alpha, seg, num_segments=n)
    amax = jnp.where(jnp.isfinite(amax), amax, 0.0)
    ex = jnp.exp(alpha - amax[seg])
    s = jax.ops.segment_sum(ex, seg, num_segments=n)
    return ex / (s[seg] + 1e-16)


def _dot_kernel(fe_ref, fn_ref, o_ref):
    o_ref[...] = jnp.sum(fe_ref[...] * fn_ref[...], axis=-1).reshape(8, 128)


def kernel(x_email, x_noun, params, edge_index_e2n, edge_index_n2e, edge_label_index):
    ei = {'e2n': edge_index_e2n, 'n2e': edge_index_n2e}
    x = {}
    x['email'] = jax.nn.relu(x_email @ params['in_email_W'] + params['in_email_b'])
    x['noun'] = jax.nn.relu(x_noun @ params['in_noun_W'] + params['in_noun_b'])
    for l in range(L):
        k, q, v = {}, {}, {}
        for t in TYPES:
            k[t] = (x[t] @ params['l%d_K_%s_W' % (l, t)] + params['l%d_K_%s_b' % (l, t)]).reshape(-1, H, DH)
            q[t] = (x[t] @ params['l%d_Q_%s_W' % (l, t)] + params['l%d_Q_%s_b' % (l, t)]).reshape(-1, H, DH)
            v[t] = (x[t] @ params['l%d_V_%s_W' % (l, t)] + params['l%d_V_%s_b' % (l, t)]).reshape(-1, H, DH)
        agg = {t: jnp.zeros((x[t].shape[0], H, DH), jnp.float32) for t in TYPES}
        for src, dst, r in RELS:
            es = ei[r][0]
            ed = ei[r][1]
            ka = jnp.einsum('nhd,hdf->nhf', k[src], params['l%d_arel_%s' % (l, r)])
            ma = jnp.einsum('nhd,hdf->nhf', v[src], params['l%d_mrel_%s' % (l, r)])
            alpha = (q[dst][ed] * ka[es]).sum(-1) * params['l%d_mu_%s' % (l, r)] / jnp.sqrt(float(DH))
            alpha = _segment_softmax(alpha, ed, x[dst].shape[0])
            agg[dst] = agg[dst] + jax.ops.segment_sum(alpha[:, :, None] * ma[es], ed, num_segments=x[dst].shape[0])
        new_x = {}
        for t in TYPES:
            o = jax.nn.gelu(agg[t].reshape(-1, D))
            o = o @ params['l%d_A_%s_W' % (l, t)] + params['l%d_A_%s_b' % (l, t)]
            beta = jax.nn.sigmoid(params['l%d_skip_%s' % (l, t)])
            new_x[t] = beta * o + (1.0 - beta) * x[t]
        x = new_x
    fe = x['email'][edge_label_index[0]]
    fn = x['noun'][edge_label_index[1]]
    EL = fe.shape[0]
    ELP = ((EL + 1023) // 1024) * 1024
    fe = jnp.pad(fe, ((0, ELP - EL), (0, 0)))
    fn = jnp.pad(fn, ((0, ELP - EL), (0, 0)))
    out = pl.pallas_call(
        _dot_kernel,
        out_shape=jax.ShapeDtypeStruct((ELP // 128, 128), jnp.float32),
        grid=(ELP // 1024,),
        in_specs=[pl.BlockSpec((1024, D), lambda i: (i, 0)),
                  pl.BlockSpec((1024, D), lambda i: (i, 0))],
        out_specs=pl.BlockSpec((8, 128), lambda i: (i, 0)),
    )(fe, fn)
    return out.reshape(-1)[:EL]


# speedup vs baseline: 7.8054x; 7.8054x over previous
import functools
import math

import jax
import jax.numpy as jnp
from jax import lax
from jax.experimental import pallas as pl
from jax.experimental.pallas import tpu as pltpu
from jax.experimental.pallas import tpu_sc as plsc

N = 50000
D = 128
H = 4
DH = D // H
L = 2
E = 400000
EL = 100000

NC = 2
NS = 16
NW = NC * NS

E_PER_W = 12800
EPAD = E_PER_W * NW
CH = 128
NCHUNK = E_PER_W // CH

DEN_PIECE = 400
AGG_PIECE = 200
NP_DEN = N // DEN_PIECE
NP_AGG = N // AGG_PIECE
K_DEN = (NP_DEN + NS - 1) // NS
K_AGG = (NP_AGG + NS - 1) // NS

ELP_PER_W = 3200
ELP = ELP_PER_W * NW

_sc_mesh = plsc.VectorSubcoreMesh(core_axis_name="c", subcore_axis_name="s")

_sc_params = pltpu.CompilerParams(
    needs_layout_passes=False, use_tc_tiling_on_sc=False)


def _fold_body(w_ref, b_ref, rel_ref, wo_ref, bo_ref):
    for h in range(H):
        r = rel_ref[h]
        sl = slice(h * DH, (h + 1) * DH)
        wo_ref[:, sl] = jnp.dot(w_ref[:, sl], r, preferred_element_type=jnp.float32)
        bo_ref[:, sl] = jnp.dot(b_ref[:, sl], r, preferred_element_type=jnp.float32)


def _fold(w, b, rel):
    return pl.pallas_call(
        _fold_body,
        out_shape=[jax.ShapeDtypeStruct((D, D), jnp.float32),
                   jax.ShapeDtypeStruct((1, D), jnp.float32)],
    )(w, b.reshape(1, D), rel)


def _inproj_body(x_ref, w_ref, b_ref, o_ref):
    o_ref[...] = jax.nn.relu(
        jnp.dot(x_ref[...], w_ref[...], preferred_element_type=jnp.float32)
        + b_ref[...])


def _inproj(x, w, b):
    BR = 400
    return pl.pallas_call(
        _inproj_body,
        out_shape=jax.ShapeDtypeStruct((N, D), jnp.float32),
        grid=(N // BR,),
        in_specs=[pl.BlockSpec((BR, D), lambda i: (i, 0)),
                  pl.BlockSpec((D, D), lambda i: (0, 0)),
                  pl.BlockSpec((1, D), lambda i: (0, 0))],
        out_specs=pl.BlockSpec((BR, D), lambda i: (i, 0)),
    )(x, w, b.reshape(1, D))


def _proj_body(x_ref, wq_ref, bq_ref, wka_ref, bka_ref, wma_ref, bma_ref,
               q_ref, ka_ref, ma_ref):
    x = x_ref[...]
    q_ref[...] = jnp.dot(x, wq_ref[...], preferred_element_type=jnp.float32) + bq_ref[...]
    ka_ref[...] = jnp.dot(x, wka_ref[...], preferred_element_type=jnp.float32) + bka_ref[...]
    ma_ref[...] = jnp.dot(x, wma_ref[...], preferred_element_type=jnp.float32) + bma_ref[...]


def _proj(x, wq, bq, wka, bka, wma, bma):
    BR = 400
    wspec = pl.BlockSpec((D, D), lambda i: (0, 0))
    bspec = pl.BlockSpec((1, D), lambda i: (0, 0))
    rspec = pl.BlockSpec((BR, D), lambda i: (i, 0))
    return pl.pallas_call(
        _proj_body,
        out_shape=[jax.ShapeDtypeStruct((N, D), jnp.float32)] * 3,
        grid=(N // BR,),
        in_specs=[rspec, wspec, bspec, wspec, bspec, wspec, bspec],
        out_specs=[rspec, rspec, rspec],
    )(x, wq, bq.reshape(1, D), wka, bka.reshape(1, D), wma, bma.reshape(1, D))


def _combine_body(p_ref, den_ref, x_ref, wa_ref, ba_ref, beta_ref, o_ref):
    p = p_ref[...]
    dens = den_ref[...]
    agg = p[:, 0] + p[:, 1]
    den = dens[0] + dens[1]
    parts = []
    for h in range(H):
        d = den[:, h:h + 1] + 1e-16
        parts.append(agg[h] / d)
    o = jnp.concatenate(parts, axis=1)
    o = jax.nn.gelu(o)
    o = jnp.dot(o, wa_ref[...], preferred_element_type=jnp.float32) + ba_ref[...]
    beta = beta_ref[0, 0]
    o_ref[...] = beta * o + (1.0 - beta) * x_ref[...]


def _combine(partials, den, x, wa, ba, beta):
    BR = 400
    return pl.pallas_call(
        _combine_body,
        out_shape=jax.ShapeDtypeStruct((N, D), jnp.float32),
        grid=(N // BR,),
        in_specs=[pl.BlockSpec((H, 2, BR, DH), lambda i: (0, 0, i, 0)),
                  pl.BlockSpec((2, BR, 16), lambda i: (0, i, 0)),
                  pl.BlockSpec((BR, D), lambda i: (i, 0)),
                  pl.BlockSpec((D, D), lambda i: (0, 0)),
                  pl.BlockSpec((1, D), lambda i: (0, 0)),
                  pl.BlockSpec((1, 1), lambda i: (0, 0))],
        out_specs=pl.BlockSpec((BR, D), lambda i: (i, 0)),
    )(partials, den, x, wa, ba.reshape(1, D), beta.reshape(1, 1))


def _passa_body(q_hbm, ka_hbm, es_hbm, ed_hbm, w_hbm, den_hbm,
                es_c, ed_c, qb, kab, obw, wout, zb, fb, den_sh, sem):
    cid = lax.axis_index("c")
    sid = lax.axis_index("s")
    wid = sid * NC + cid
    iota = lax.iota(jnp.int32, 16)
    zeros16 = jnp.zeros((16,), jnp.float32)

    def _zrow(r, _):
        zb[r, pl.ds(0, 16)] = zeros16
        return 0
    lax.fori_loop(0, DEN_PIECE, _zrow, 0)

    def _orow(r, _):
        obw[r, pl.ds(0, 16)] = zeros16
        return 0
    lax.fori_loop(0, CH, _orow, 0)

    for k in range(K_DEN):
        piece = sid + NS * k
        @pl.when(piece < NP_DEN)
        def _():
            pltpu.sync_copy(zb, den_sh.at[pl.ds(piece * DEN_PIECE, DEN_PIECE)])
    plsc.subcore_barrier()

    def _chunk(c, _):
        gbase = wid * E_PER_W + c * CH
        pltpu.sync_copy(es_hbm.at[pl.ds(gbase, CH)], es_c)
        pltpu.sync_copy(ed_hbm.at[pl.ds(gbase, CH)], ed_c)
        pltpu.async_copy(q_hbm.at[ed_c], qb, sem).wait()
        pltpu.async_copy(ka_hbm.at[es_c], kab, sem).wait()

        def _grp(g, _):
            rows = iota + g * 16
            gid = gbase + g * 16 + iota
            for h in range(H):
                def _dot(d, acc):
                    dcol = jnp.full((16,), h * DH + d, jnp.int32)
                    qv = plsc.load_gather(qb, [rows, dcol])
                    kv = plsc.load_gather(kab, [rows, dcol])
                    return acc + qv * kv
                acc = lax.fori_loop(0, DH, _dot, jnp.zeros((16,), jnp.float32))
                w = jnp.exp(acc)
                w = jnp.where(gid < E, w, 0.0)
                hcol = jnp.full((16,), h, jnp.int32)
                plsc.store_scatter(obw, [rows, hcol], w)
                plsc.store_scatter(wout, [rows, hcol], w)
            return 0
        lax.fori_loop(0, CH // 16, _grp, 0)

        pltpu.sync_copy(wout, w_hbm.at[pl.ds(gbase, CH)])
        pltpu.sync_copy(obw, den_sh.at[ed_c], add=True)
        return 0
    lax.fori_loop(0, NCHUNK, _chunk, 0)
    plsc.subcore_barrier()

    for k in range(K_DEN):
        piece = sid + NS * k
        @pl.when(piece < NP_DEN)
        def _():
            start = piece * DEN_PIECE
            pltpu.sync_copy(den_sh.at[pl.ds(start, DEN_PIECE)], fb)
            pltpu.sync_copy(fb, den_hbm.at[cid, pl.ds(start, DEN_PIECE)])


_passa_call = functools.partial(
    pl.kernel,
    _passa_body,
    out_type=[jax.ShapeDtypeStruct((EPAD, 4), jnp.float32),
              jax.ShapeDtypeStruct((NC, N, 16), jnp.float32)],
    mesh=_sc_mesh,
    compiler_params=_sc_params,
    scratch_types=[
        pltpu.VMEM((CH,), jnp.int32),
        pltpu.VMEM((CH,), jnp.int32),
        pltpu.VMEM((CH, D), jnp.float32),
        pltpu.VMEM((CH, D), jnp.float32),
        pltpu.VMEM((CH, 16), jnp.float32),
        pltpu.VMEM((CH, 4), jnp.float32),
        pltpu.VMEM((DEN_PIECE, 16), jnp.float32),
        pltpu.VMEM((DEN_PIECE, 16), jnp.float32),
        pltpu.VMEM_SHARED((N, 16), jnp.float32),
        pltpu.SemaphoreType.DMA,
    ],
)()


def _passb_body(ma_hbm, w_hbm, es_hbm, ed_hbm, out_hbm,
                es_c, ed_c, esg, wb, mab, ob, zb, fb, agg_sh, sem):
    cid = lax.axis_index("c")
    sid = lax.axis_index("s")
    wid = sid * NC + cid
    iota = lax.iota(jnp.int32, 16)
    zeros16 = jnp.zeros((16,), jnp.float32)

    def _zrow(r, _):
        for c0 in (0, 16):
            zb[r, pl.ds(c0, 16)] = zeros16
        return 0
    lax.fori_loop(0, AGG_PIECE, _zrow, 0)

    for h in range(H):
        for k in range(K_AGG):
            piece = sid + NS * k
            @pl.when(piece < NP_AGG)
            def _():
                pltpu.sync_copy(zb, agg_sh.at[pl.ds(piece * AGG_PIECE, AGG_PIECE)])
        plsc.subcore_barrier()

        def _chunk(c, _):
            gbase = wid * E_PER_W + c * CH
            pltpu.sync_copy(es_hbm.at[pl.ds(gbase, CH)], es_c)
            pltpu.sync_copy(ed_hbm.at[pl.ds(gbase, CH)], ed_c)
            pltpu.sync_copy(w_hbm.at[pl.ds(gbase, CH)], wb)

            def _gidx(g, _):
                ev = es_c[pl.ds(g * 16, 16)]
                esg[pl.ds(g * 16, 16)] = ev * H + h
                return 0
            lax.fori_loop(0, CH // 16, _gidx, 0)

            pltpu.async_copy(ma_hbm.at[esg], mab, sem).wait()

            def _grp(g, _):
                rows = iota + g * 16
                hcol = jnp.full((16,), h, jnp.int32)
                w = plsc.load_gather(wb, [rows, hcol])

                def _mrow(d, _):
                    dcol = jnp.full((16,), d, jnp.int32)
                    mv = plsc.load_gather(mab, [rows, dcol])
                    plsc.store_scatter(ob, [rows, dcol], mv * w)
                    return 0
                lax.fori_loop(0, DH, _mrow, 0)
                return 0
            lax.fori_loop(0, CH // 16, _grp, 0)

            pltpu.sync_copy(ob, agg_sh.at[ed_c], add=True)
            return 0
        lax.fori_loop(0, NCHUNK, _chunk, 0)
        plsc.subcore_barrier()

        for k in range(K_AGG):
            piece = sid + NS * k
            @pl.when(piece < NP_AGG)
            def _():
                start = piece * AGG_PIECE
                pltpu.sync_copy(agg_sh.at[pl.ds(start, AGG_PIECE)], fb)
                pltpu.sync_copy(fb, out_hbm.at[h, cid, pl.ds(start, AGG_PIECE)])
        plsc.subcore_barrier()


_passb_call = functools.partial(
    pl.kernel,
    _passb_body,
    out_type=jax.ShapeDtypeStruct((H, NC, N, DH), jnp.float32),
    mesh=_sc_mesh,
    compiler_params=_sc_params,
    scratch_types=[
        pltpu.VMEM((CH,), jnp.int32),
        pltpu.VMEM((CH,), jnp.int32),
        pltpu.VMEM((CH,), jnp.int32),
        pltpu.VMEM((CH, 4), jnp.float32),
        pltpu.VMEM((CH, DH), jnp.float32),
        pltpu.VMEM((CH, DH), jnp.float32),
        pltpu.VMEM((AGG_PIECE, DH), jnp.float32),
        pltpu.VMEM((AGG_PIECE, DH), jnp.float32),
        pltpu.VMEM_SHARED((N, DH), jnp.float32),
        pltpu.SemaphoreType.DMA,
    ],
)()


def _score_body(xe_hbm, xn_hbm, i0_hbm, i1_hbm, out_hbm,
                ia, ib, feb, fnb, outb, sem):
    cid = lax.axis_index("c")
    sid = lax.axis_index("s")
    wid = sid * NC + cid
    iota = lax.iota(jnp.int32, 16)

    def _chunk(c, _):
        gbase = wid * ELP_PER_W + c * CH
        pltpu.sync_copy(i0_hbm.at[pl.ds(gbase, CH)], ia)
        pltpu.sync_copy(i1_hbm.at[pl.ds(gbase, CH)], ib)
        pltpu.async_copy(xe_hbm.at[ia], feb, sem).wait()
        pltpu.async_copy(xn_hbm.at[ib], fnb, sem).wait()

        def _grp(g, _):
            rows = iota + g * 16

            def _dot(d, acc):
                dcol = jnp.full((16,), d, jnp.int32)
                fv = plsc.load_gather(feb, [rows, dcol])
                nv = plsc.load_gather(fnb, [rows, dcol])
                return acc + fv * nv
            acc = lax.fori_loop(0, D, _dot, jnp.zeros((16,), jnp.float32))
            outb[pl.ds(g * 16, 16)] = acc
            return 0
        lax.fori_loop(0, CH // 16, _grp, 0)

        pltpu.sync_copy(outb, out_hbm.at[pl.ds(gbase, CH)])
        return 0
    lax.fori_loop(0, ELP_PER_W // CH, _chunk, 0)


_score_call = functools.partial(
    pl.kernel,
    _score_body,
    out_type=jax.ShapeDtypeStruct((ELP,), jnp.float32),
    mesh=_sc_mesh,
    compiler_params=_sc_params,
    scratch_types=[
        pltpu.VMEM((CH,), jnp.int32),
        pltpu.VMEM((CH,), jnp.int32),
        pltpu.VMEM((CH, D), jnp.float32),
        pltpu.VMEM((CH, D), jnp.float32),
        pltpu.VMEM((CH,), jnp.float32),
        pltpu.SemaphoreType.DMA,
    ],
)()


RELS = [('email', 'noun', 'e2n'), ('noun', 'email', 'n2e')]


def kernel(x_email, x_noun, params, edge_index_e2n, edge_index_n2e, edge_label_index):
    p = params
    ei = {'e2n': edge_index_e2n, 'n2e': edge_index_n2e}
    es_pad, ed_pad = {}, {}
    for r in ('e2n', 'n2e'):
        es_pad[r] = jnp.pad(ei[r][0], (0, EPAD - E))
        ed_pad[r] = jnp.pad(ei[r][1], (0, EPAD - E))

    x = {
        'email': _inproj(x_email, p['in_email_W'], p['in_email_b']),
        'noun': _inproj(x_noun, p['in_noun_W'], p['in_noun_b']),
    }

    inv_sqrt_dh = 1.0 / math.sqrt(float(DH))
    for l in range(L):
        folded = {}
        for src, dst, r in RELS:
            arel_s = p['l%d_arel_%s' % (l, r)] * (
                p['l%d_mu_%s' % (l, r)] * inv_sqrt_dh)[:, None, None]
            wka, bka = _fold(p['l%d_K_%s_W' % (l, src)],
                             p['l%d_K_%s_b' % (l, src)], arel_s)
            wma, bma = _fold(p['l%d_V_%s_W' % (l, src)],
                             p['l%d_V_%s_b' % (l, src)],
                             p['l%d_mrel_%s' % (l, r)])
            folded[src] = (wka, bka, wma, bma)

        tabs = {}
        for t in ('email', 'noun'):
            wka, bka, wma, bma = folded[t]
            q, ka, ma = _proj(x[t],
                              p['l%d_Q_%s_W' % (l, t)], p['l%d_Q_%s_b' % (l, t)],
                              wka, bka.reshape(D), wma, bma.reshape(D))
            tabs[t] = (q, ka, ma.reshape(N * H, DH))

        new_x = {}
        for src, dst, r in RELS:
            q_dst = tabs[dst][0]
            ka_src = tabs[src][1]
            ma_src = tabs[src][2]
            w_e, den = _passa_call(q_dst, ka_src, es_pad[r], ed_pad[r])
            partials = _passb_call(ma_src, w_e, es_pad[r], ed_pad[r])
            beta = jax.nn.sigmoid(p['l%d_skip_%s' % (l, dst)])
            new_x[dst] = _combine(partials, den, x[dst],
                                  p['l%d_A_%s_W' % (l, dst)],
                                  p['l%d_A_%s_b' % (l, dst)], beta)
        x = new_x

    eli0 = jnp.pad(edge_label_index[0], (0, ELP - EL))
    eli1 = jnp.pad(edge_label_index[1], (0, ELP - EL))
    out = _score_call(x['email'], x['noun'], eli0, eli1)
    return out[:EL]


# speedup vs baseline: 11.5179x; 1.4756x over previous
import functools
import math

import jax
import jax.numpy as jnp
from jax import lax
from jax.experimental import pallas as pl
from jax.experimental.pallas import tpu as pltpu
from jax.experimental.pallas import tpu_sc as plsc

N = 50000
D = 128
H = 4
DH = D // H
L = 2
E = 400000
EL = 100000

NC = 2
NS = 16
NW = NC * NS

E_PER_W = 12800
EPAD = E_PER_W * NW
CH = 128
NCHUNK = E_PER_W // CH

DEN_PIECE = 200
AGG_PIECE = 200
NP_DEN = N // DEN_PIECE
NP_AGG = N // AGG_PIECE
K_DEN = (NP_DEN + NS - 1) // NS
K_AGG = (NP_AGG + NS - 1) // NS

ELP_PER_W = 3328
ELP = ELP_PER_W * NW
NCHUNK_S = ELP_PER_W // CH

_sc_mesh = plsc.VectorSubcoreMesh(core_axis_name="c", subcore_axis_name="s")

_sc_params = pltpu.CompilerParams(
    needs_layout_passes=False, use_tc_tiling_on_sc=False)


def _fold_body(w_ref, b_ref, rel_ref, wo_ref, bo_ref):
    for h in range(H):
        r = rel_ref[h]
        sl = slice(h * DH, (h + 1) * DH)
        wo_ref[:, sl] = jnp.dot(w_ref[:, sl], r, preferred_element_type=jnp.float32)
        bo_ref[:, sl] = jnp.dot(b_ref[:, sl], r, preferred_element_type=jnp.float32)


def _fold(w, b, rel):
    return pl.pallas_call(
        _fold_body,
        out_shape=[jax.ShapeDtypeStruct((D, D), jnp.float32),
                   jax.ShapeDtypeStruct((1, D), jnp.float32)],
    )(w, b.reshape(1, D), rel)


def _inproj_body(x_ref, w_ref, b_ref, o_ref):
    o_ref[...] = jax.nn.relu(
        jnp.dot(x_ref[...], w_ref[...], preferred_element_type=jnp.float32)
        + b_ref[...])


def _inproj(x, w, b):
    BR = 400
    return pl.pallas_call(
        _inproj_body,
        out_shape=jax.ShapeDtypeStruct((N, D), jnp.float32),
        grid=(N // BR,),
        in_specs=[pl.BlockSpec((BR, D), lambda i: (i, 0)),
                  pl.BlockSpec((D, D), lambda i: (0, 0)),
                  pl.BlockSpec((1, D), lambda i: (0, 0))],
        out_specs=pl.BlockSpec((BR, D), lambda i: (i, 0)),
    )(x, w, b.reshape(1, D))


def _proj_body(x_ref, wq_ref, bq_ref, wka_ref, bka_ref, wma_ref, bma_ref,
               q_ref, ka_ref, ma_ref):
    x = x_ref[...]
    q_ref[...] = jnp.dot(x, wq_ref[...], preferred_element_type=jnp.float32) + bq_ref[...]
    ka_ref[...] = jnp.dot(x, wka_ref[...], preferred_element_type=jnp.float32) + bka_ref[...]
    ma_ref[...] = jnp.dot(x, wma_ref[...], preferred_element_type=jnp.float32) + bma_ref[...]


def _proj(x, wq, bq, wka, bka, wma, bma):
    BR = 400
    wspec = pl.BlockSpec((D, D), lambda i: (0, 0))
    bspec = pl.BlockSpec((1, D), lambda i: (0, 0))
    rspec = pl.BlockSpec((BR, D), lambda i: (i, 0))
    return pl.pallas_call(
        _proj_body,
        out_shape=[jax.ShapeDtypeStruct((N, D), jnp.float32)] * 3,
        grid=(N // BR,),
        in_specs=[rspec, wspec, bspec, wspec, bspec, wspec, bspec],
        out_specs=[rspec, rspec, rspec],
    )(x, wq, bq.reshape(1, D), wka, bka.reshape(1, D), wma, bma.reshape(1, D))


def _combine_body(p_ref, den_ref, x_ref, wa_ref, ba_ref, beta_ref, o_ref):
    p = p_ref[...]
    dens = den_ref[...]
    agg = p[:, 0] + p[:, 1]
    den = dens[0] + dens[1]
    parts = []
    for h in range(H):
        d = den[:, h:h + 1] + 1e-16
        parts.append(agg[h] / d)
    o = jnp.concatenate(parts, axis=1)
    o = jax.nn.gelu(o)
    o = jnp.dot(o, wa_ref[...], preferred_element_type=jnp.float32) + ba_ref[...]
    beta = beta_ref[0, 0]
    o_ref[...] = beta * o + (1.0 - beta) * x_ref[...]


def _combine(partials, den, x, wa, ba, beta):
    BR = 400
    return pl.pallas_call(
        _combine_body,
        out_shape=jax.ShapeDtypeStruct((N, D), jnp.float32),
        grid=(N // BR,),
        in_specs=[pl.BlockSpec((H, 2, BR, DH), lambda i: (0, 0, i, 0)),
                  pl.BlockSpec((2, BR, 16), lambda i: (0, i, 0)),
                  pl.BlockSpec((BR, D), lambda i: (i, 0)),
                  pl.BlockSpec((D, D), lambda i: (0, 0)),
                  pl.BlockSpec((1, D), lambda i: (0, 0)),
                  pl.BlockSpec((1, 1), lambda i: (0, 0))],
        out_specs=pl.BlockSpec((BR, D), lambda i: (i, 0)),
    )(partials, den, x, wa, ba.reshape(1, D), beta.reshape(1, 1))


def _passa_body(q_hbm, ka_hbm, es_hbm, ed_hbm, w_hbm, den_hbm,
                es0, ed0, es1, ed1, qb0, kab0, qb1, kab1, obw, wout,
                zb, fb, den_sh, si0, si1, sg0, sg1):
    cid = lax.axis_index("c")
    sid = lax.axis_index("s")
    wid = sid * NC + cid
    iota = lax.iota(jnp.int32, 16)
    zeros16 = jnp.zeros((16,), jnp.float32)
    esb = (es0, es1)
    edb = (ed0, ed1)
    qbb = (qb0, qb1)
    kbb = (kab0, kab1)
    sib = (si0, si1)
    sgb = (sg0, sg1)

    def _gbase(c):
        return wid * E_PER_W + c * CH

    def issue_idx(s, c):
        pltpu.async_copy(es_hbm.at[pl.ds(_gbase(c), CH)], esb[s], sib[s])
        pltpu.async_copy(ed_hbm.at[pl.ds(_gbase(c), CH)], edb[s], sib[s])

    def wait_idx(s, c):
        pltpu.make_async_copy(es_hbm.at[pl.ds(_gbase(c), CH)], esb[s], sib[s]).wait()
        pltpu.make_async_copy(ed_hbm.at[pl.ds(_gbase(c), CH)], edb[s], sib[s]).wait()

    def issue_gather(s):
        pltpu.async_copy(q_hbm.at[edb[s]], qbb[s], sgb[s])
        pltpu.async_copy(ka_hbm.at[esb[s]], kbb[s], sgb[s])

    def wait_gather(s):
        pltpu.make_async_copy(q_hbm.at[edb[s]], qbb[s], sgb[s]).wait()
        pltpu.make_async_copy(ka_hbm.at[esb[s]], kbb[s], sgb[s]).wait()

    def _zrow(r, _):
        zb[r, pl.ds(0, 16)] = zeros16
        return 0
    lax.fori_loop(0, DEN_PIECE, _zrow, 0)

    def _orow(r, _):
        obw[r, pl.ds(0, 16)] = zeros16
        return 0
    lax.fori_loop(0, CH, _orow, 0)

    for k in range(K_DEN):
        piece = sid + NS * k
        @pl.when(piece < NP_DEN)
        def _():
            pltpu.sync_copy(zb, den_sh.at[pl.ds(piece * DEN_PIECE, DEN_PIECE)])
    plsc.subcore_barrier()

    issue_idx(0, 0)
    wait_idx(0, 0)
    issue_gather(0)
    issue_idx(1, 1)

    def compute(s, c):
        qb = qbb[s]
        kab = kbb[s]

        def _grp(g, _):
            rows = iota + g * 16
            gid = _gbase(c) + g * 16 + iota
            for h in range(H):
                acc = jnp.zeros((16,), jnp.float32)
                for d in range(DH):
                    dcol = jnp.full((16,), h * DH + d, jnp.int32)
                    acc = acc + (plsc.load_gather(qb, [rows, dcol])
                                 * plsc.load_gather(kab, [rows, dcol]))
                w = jnp.exp(acc)
                w = jnp.where(gid < E, w, 0.0)
                hcol = jnp.full((16,), h, jnp.int32)
                plsc.store_scatter(obw, [rows, hcol], w)
                plsc.store_scatter(wout, [rows, hcol], w)
            return 0
        lax.fori_loop(0, CH // 16, _grp, 0)
        pltpu.sync_copy(wout, w_hbm.at[pl.ds(_gbase(c), CH)])
        pltpu.sync_copy(obw, den_sh.at[edb[s]], add=True)

    def _iter(c2, _):
        for par in (0, 1):
            c = 2 * c2 + par
            o = 1 - par

            @pl.when(c + 1 < NCHUNK)
            def _():
                wait_idx(o, c + 1)
                issue_gather(o)
            wait_gather(par)
            compute(par, c)

            @pl.when(c + 2 < NCHUNK)
            def _():
                issue_idx(par, c + 2)
        return 0
    lax.fori_loop(0, NCHUNK // 2, _iter, 0)
    plsc.subcore_barrier()

    for k in range(K_DEN):
        piece = sid + NS * k
        @pl.when(piece < NP_DEN)
        def _():
            start = piece * DEN_PIECE
            pltpu.sync_copy(den_sh.at[pl.ds(start, DEN_PIECE)], fb)
            pltpu.sync_copy(fb, den_hbm.at[cid, pl.ds(start, DEN_PIECE)])


_passa_call = functools.partial(
    pl.kernel,
    _passa_body,
    out_type=[jax.ShapeDtypeStruct((EPAD, 4), jnp.float32),
              jax.ShapeDtypeStruct((NC, N, 16), jnp.float32)],
    mesh=_sc_mesh,
    compiler_params=_sc_params,
    scratch_types=[
        pltpu.VMEM((CH,), jnp.int32),
        pltpu.VMEM((CH,), jnp.int32),
        pltpu.VMEM((CH,), jnp.int32),
        pltpu.VMEM((CH,), jnp.int32),
        pltpu.VMEM((CH, D), jnp.float32),
        pltpu.VMEM((CH, D), jnp.float32),
        pltpu.VMEM((CH, D), jnp.float32),
        pltpu.VMEM((CH, D), jnp.float32),
        pltpu.VMEM((CH, 16), jnp.float32),
        pltpu.VMEM((CH, 4), jnp.float32),
        pltpu.VMEM((DEN_PIECE, 16), jnp.float32),
        pltpu.VMEM((DEN_PIECE, 16), jnp.float32),
        pltpu.VMEM_SHARED((N, 16), jnp.float32),
        pltpu.SemaphoreType.DMA,
        pltpu.SemaphoreType.DMA,
        pltpu.SemaphoreType.DMA,
        pltpu.SemaphoreType.DMA,
    ],
)()


def _passb_body(ma_hbm, w_hbm, es_hbm, ed_hbm, out_hbm,
                es0, ed0, esg0, wb0, mab0, es1, ed1, esg1, wb1, mab1,
                ob, zb, fb, agg_sh, si0, si1, sg0, sg1):
    cid = lax.axis_index("c")
    sid = lax.axis_index("s")
    wid = sid * NC + cid
    iota = lax.iota(jnp.int32, 16)
    zeros16 = jnp.zeros((16,), jnp.float32)
    esb = (es0, es1)
    edb = (ed0, ed1)
    egb = (esg0, esg1)
    wbb = (wb0, wb1)
    mbb = (mab0, mab1)
    sib = (si0, si1)
    sgb = (sg0, sg1)

    def _gbase(c):
        return wid * E_PER_W + c * CH

    def issue_idx(s, c):
        pltpu.async_copy(es_hbm.at[pl.ds(_gbase(c), CH)], esb[s], sib[s])
        pltpu.async_copy(ed_hbm.at[pl.ds(_gbase(c), CH)], edb[s], sib[s])
        pltpu.async_copy(w_hbm.at[pl.ds(_gbase(c), CH)], wbb[s], sib[s])

    def wait_idx(s, c):
        pltpu.make_async_copy(es_hbm.at[pl.ds(_gbase(c), CH)], esb[s], sib[s]).wait()
        pltpu.make_async_copy(ed_hbm.at[pl.ds(_gbase(c), CH)], edb[s], sib[s]).wait()
        pltpu.make_async_copy(w_hbm.at[pl.ds(_gbase(c), CH)], wbb[s], sib[s]).wait()

    def _zrow(r, _):
        for c0 in (0, 16):
            zb[r, pl.ds(c0, 16)] = zeros16
        return 0
    lax.fori_loop(0, AGG_PIECE, _zrow, 0)

    for h in range(H):
        def prep_gather(s, h=h):
            def _gidx(g, _):
                ev = esb[s][pl.ds(g * 16, 16)]
                egb[s][pl.ds(g * 16, 16)] = ev * H + h
                return 0
            lax.fori_loop(0, CH // 16, _gidx, 0)
            pltpu.async_copy(ma_hbm.at[egb[s]], mbb[s], sgb[s])

        def wait_gather(s):
            pltpu.make_async_copy(ma_hbm.at[egb[s]], mbb[s], sgb[s]).wait()

        for k in range(K_AGG):
            piece = sid + NS * k
            @pl.when(piece < NP_AGG)
            def _():
                pltpu.sync_copy(zb, agg_sh.at[pl.ds(piece * AGG_PIECE, AGG_PIECE)])
        plsc.subcore_barrier()

        issue_idx(0, 0)
        wait_idx(0, 0)
        prep_gather(0)
        issue_idx(1, 1)

        def compute(s, h=h):
            mab = mbb[s]

            def _grp(g, _):
                rows = iota + g * 16
                hcol = jnp.full((16,), h, jnp.int32)
                w = plsc.load_gather(wbb[s], [rows, hcol])
                for d in range(DH):
                    dcol = jnp.full((16,), d, jnp.int32)
                    mv = plsc.load_gather(mab, [rows, dcol])
                    plsc.store_scatter(ob, [rows, dcol], mv * w)
                return 0
            lax.fori_loop(0, CH // 16, _grp, 0)
            pltpu.sync_copy(ob, agg_sh.at[edb[s]], add=True)

        def _iter(c2, _):
            for par in (0, 1):
                c = 2 * c2 + par
                o = 1 - par

                @pl.when(c + 1 < NCHUNK)
                def _():
                    wait_idx(o, c + 1)
                    prep_gather(o)
                wait_gather(par)
                compute(par)

                @pl.when(c + 2 < NCHUNK)
                def _():
                    issue_idx(par, c + 2)
            return 0
        lax.fori_loop(0, NCHUNK // 2, _iter, 0)
        plsc.subcore_barrier()

        for k in range(K_AGG):
            piece = sid + NS * k
            @pl.when(piece < NP_AGG)
            def _():
                start = piece * AGG_PIECE
                pltpu.sync_copy(agg_sh.at[pl.ds(start, AGG_PIECE)], fb)
                pltpu.sync_copy(fb, out_hbm.at[h, cid, pl.ds(start, AGG_PIECE)])
        plsc.subcore_barrier()


_passb_call = functools.partial(
    pl.kernel,
    _passb_body,
    out_type=jax.ShapeDtypeStruct((H, NC, N, DH), jnp.float32),
    mesh=_sc_mesh,
    compiler_params=_sc_params,
    scratch_types=[
        pltpu.VMEM((CH,), jnp.int32),
        pltpu.VMEM((CH,), jnp.int32),
        pltpu.VMEM((CH,), jnp.int32),
        pltpu.VMEM((CH, 4), jnp.float32),
        pltpu.VMEM((CH, DH), jnp.float32),
        pltpu.VMEM((CH,), jnp.int32),
        pltpu.VMEM((CH,), jnp.int32),
        pltpu.VMEM((CH,), jnp.int32),
        pltpu.VMEM((CH, 4), jnp.float32),
        pltpu.VMEM((CH, DH), jnp.float32),
        pltpu.VMEM((CH, DH), jnp.float32),
        pltpu.VMEM((AGG_PIECE, DH), jnp.float32),
        pltpu.VMEM((AGG_PIECE, DH), jnp.float32),
        pltpu.VMEM_SHARED((N, DH), jnp.float32),
        pltpu.SemaphoreType.DMA,
        pltpu.SemaphoreType.DMA,
        pltpu.SemaphoreType.DMA,
        pltpu.SemaphoreType.DMA,
    ],
)()


def _score_body(xe_hbm, xn_hbm, i0_hbm, i1_hbm, out_hbm,
                ia0, ib0, ia1, ib1, feb0, fnb0, feb1, fnb1, outb,
                si0, si1, sg0, sg1):
    cid = lax.axis_index("c")
    sid = lax.axis_index("s")
    wid = sid * NC + cid
    iota = lax.iota(jnp.int32, 16)
    iab = (ia0, ia1)
    ibb = (ib0, ib1)
    feb = (feb0, feb1)
    fnb = (fnb0, fnb1)
    sib = (si0, si1)
    sgb = (sg0, sg1)

    def _gbase(c):
        return wid * ELP_PER_W + c * CH

    def issue_idx(s, c):
        pltpu.async_copy(i0_hbm.at[pl.ds(_gbase(c), CH)], iab[s], sib[s])
        pltpu.async_copy(i1_hbm.at[pl.ds(_gbase(c), CH)], ibb[s], sib[s])

    def wait_idx(s, c):
        pltpu.make_async_copy(i0_hbm.at[pl.ds(_gbase(c), CH)], iab[s], sib[s]).wait()
        pltpu.make_async_copy(i1_hbm.at[pl.ds(_gbase(c), CH)], ibb[s], sib[s]).wait()

    def issue_gather(s):
        pltpu.async_copy(xe_hbm.at[iab[s]], feb[s], sgb[s])
        pltpu.async_copy(xn_hbm.at[ibb[s]], fnb[s], sgb[s])

    def wait_gather(s):
        pltpu.make_async_copy(xe_hbm.at[iab[s]], feb[s], sgb[s]).wait()
        pltpu.make_async_copy(xn_hbm.at[ibb[s]], fnb[s], sgb[s]).wait()

    issue_idx(0, 0)
    wait_idx(0, 0)
    issue_gather(0)
    issue_idx(1, 1)

    def compute(s, c):
        def _grp(g, _):
            rows = iota + g * 16
            acc = jnp.zeros((16,), jnp.float32)
            for d in range(D):
                dcol = jnp.full((16,), d, jnp.int32)
                acc = acc + (plsc.load_gather(feb[s], [rows, dcol])
                             * plsc.load_gather(fnb[s], [rows, dcol]))
            outb[pl.ds(g * 16, 16)] = acc
            return 0
        lax.fori_loop(0, CH // 16, _grp, 0)
        pltpu.sync_copy(outb, out_hbm.at[pl.ds(_gbase(c), CH)])

    def _iter(c2, _):
        for par in (0, 1):
            c = 2 * c2 + par
            o = 1 - par

            @pl.when(c + 1 < NCHUNK_S)
            def _():
                wait_idx(o, c + 1)
                issue_gather(o)
            wait_gather(par)
            compute(par, c)

            @pl.when(c + 2 < NCHUNK_S)
            def _():
                issue_idx(par, c + 2)
        return 0
    lax.fori_loop(0, NCHUNK_S // 2, _iter, 0)


_score_call = functools.partial(
    pl.kernel,
    _score_body,
    out_type=jax.ShapeDtypeStruct((ELP,), jnp.float32),
    mesh=_sc_mesh,
    compiler_params=_sc_params,
    scratch_types=[
        pltpu.VMEM((CH,), jnp.int32),
        pltpu.VMEM((CH,), jnp.int32),
        pltpu.VMEM((CH,), jnp.int32),
        pltpu.VMEM((CH,), jnp.int32),
        pltpu.VMEM((CH, D), jnp.float32),
        pltpu.VMEM((CH, D), jnp.float32),
        pltpu.VMEM((CH, D), jnp.float32),
        pltpu.VMEM((CH, D), jnp.float32),
        pltpu.VMEM((CH,), jnp.float32),
        pltpu.SemaphoreType.DMA,
        pltpu.SemaphoreType.DMA,
        pltpu.SemaphoreType.DMA,
        pltpu.SemaphoreType.DMA,
    ],
)()


RELS = [('email', 'noun', 'e2n'), ('noun', 'email', 'n2e')]


def kernel(x_email, x_noun, params, edge_index_e2n, edge_index_n2e, edge_label_index):
    p = params
    ei = {'e2n': edge_index_e2n, 'n2e': edge_index_n2e}
    es_pad, ed_pad = {}, {}
    for r in ('e2n', 'n2e'):
        es_pad[r] = jnp.pad(ei[r][0], (0, EPAD - E))
        ed_pad[r] = jnp.pad(ei[r][1], (0, EPAD - E))

    x = {
        'email': _inproj(x_email, p['in_email_W'], p['in_email_b']),
        'noun': _inproj(x_noun, p['in_noun_W'], p['in_noun_b']),
    }

    inv_sqrt_dh = 1.0 / math.sqrt(float(DH))
    for l in range(L):
        folded = {}
        for src, dst, r in RELS:
            arel_s = p['l%d_arel_%s' % (l, r)] * (
                p['l%d_mu_%s' % (l, r)] * inv_sqrt_dh)[:, None, None]
            wka, bka = _fold(p['l%d_K_%s_W' % (l, src)],
                             p['l%d_K_%s_b' % (l, src)], arel_s)
            wma, bma = _fold(p['l%d_V_%s_W' % (l, src)],
                             p['l%d_V_%s_b' % (l, src)],
                             p['l%d_mrel_%s' % (l, r)])
            folded[src] = (wka, bka, wma, bma)

        tabs = {}
        for t in ('email', 'noun'):
            wka, bka, wma, bma = folded[t]
            q, ka, ma = _proj(x[t],
                              p['l%d_Q_%s_W' % (l, t)], p['l%d_Q_%s_b' % (l, t)],
                              wka, bka.reshape(D), wma, bma.reshape(D))
            tabs[t] = (q, ka, ma.reshape(N * H, DH))

        new_x = {}
        for src, dst, r in RELS:
            q_dst = tabs[dst][0]
            ka_src = tabs[src][1]
            ma_src = tabs[src][2]
            w_e, den = _passa_call(q_dst, ka_src, es_pad[r], ed_pad[r])
            partials = _passb_call(ma_src, w_e, es_pad[r], ed_pad[r])
            beta = jax.nn.sigmoid(p['l%d_skip_%s' % (l, dst)])
            new_x[dst] = _combine(partials, den, x[dst],
                                  p['l%d_A_%s_W' % (l, dst)],
                                  p['l%d_A_%s_b' % (l, dst)], beta)
        x = new_x

    eli0 = jnp.pad(edge_label_index[0], (0, ELP - EL))
    eli1 = jnp.pad(edge_label_index[1], (0, ELP - EL))
    out = _score_call(x['email'], x['noun'], eli0, eli1)
    return out[:EL]


# speedup vs baseline: 25.1183x; 2.1808x over previous
import functools
import math

import jax
import jax.numpy as jnp
from jax import lax
from jax.experimental import pallas as pl
from jax.experimental.pallas import tpu as pltpu
from jax.experimental.pallas import tpu_sc as plsc

N = 50000
D = 128
H = 4
DH = D // H
L = 2
E = 400000
EL = 100000

NC = 2
NS = 16
NW = NC * NS

E_PER_W = 12800
EPAD = E_PER_W * NW
CH = 128
NCHUNK = E_PER_W // CH

DEN_PIECE = 200
AGG_PIECE = 200
NP_DEN = N // DEN_PIECE
NP_AGG = N // AGG_PIECE
K_DEN = (NP_DEN + NS - 1) // NS
K_AGG = (NP_AGG + NS - 1) // NS

ELP_PER_W = 3328
ELP = ELP_PER_W * NW
NCHUNK_S = ELP_PER_W // CH

_sc_mesh = plsc.VectorSubcoreMesh(core_axis_name="c", subcore_axis_name="s")

_sc_params = pltpu.CompilerParams(
    needs_layout_passes=False, use_tc_tiling_on_sc=False)


def _fold_body(w_ref, b_ref, rel_ref, wo_ref, bo_ref):
    for h in range(H):
        r = rel_ref[h]
        sl = slice(h * DH, (h + 1) * DH)
        wo_ref[:, sl] = jnp.dot(w_ref[:, sl], r, preferred_element_type=jnp.float32)
        bo_ref[:, sl] = jnp.dot(b_ref[:, sl], r, preferred_element_type=jnp.float32)


def _fold(w, b, rel):
    return pl.pallas_call(
        _fold_body,
        out_shape=[jax.ShapeDtypeStruct((D, D), jnp.float32),
                   jax.ShapeDtypeStruct((1, D), jnp.float32)],
    )(w, b.reshape(1, D), rel)


def _inproj_body(x_ref, w_ref, b_ref, o_ref):
    o_ref[...] = jax.nn.relu(
        jnp.dot(x_ref[...], w_ref[...], preferred_element_type=jnp.float32)
        + b_ref[...])


def _inproj(x, w, b):
    BR = 400
    return pl.pallas_call(
        _inproj_body,
        out_shape=jax.ShapeDtypeStruct((N, D), jnp.float32),
        grid=(N // BR,),
        in_specs=[pl.BlockSpec((BR, D), lambda i: (i, 0)),
                  pl.BlockSpec((D, D), lambda i: (0, 0)),
                  pl.BlockSpec((1, D), lambda i: (0, 0))],
        out_specs=pl.BlockSpec((BR, D), lambda i: (i, 0)),
    )(x, w, b.reshape(1, D))


def _proj_body(x_ref, wq_ref, bq_ref, wka_ref, bka_ref, wma_ref, bma_ref,
               q_ref, ka_ref, ma_ref):
    x = x_ref[...]
    q_ref[...] = jnp.dot(x, wq_ref[...], preferred_element_type=jnp.float32) + bq_ref[...]
    ka_ref[...] = jnp.dot(x, wka_ref[...], preferred_element_type=jnp.float32) + bka_ref[...]
    ma_ref[...] = jnp.dot(x, wma_ref[...], preferred_element_type=jnp.float32) + bma_ref[...]


def _proj(x, wq, bq, wka, bka, wma, bma):
    BR = 400
    wspec = pl.BlockSpec((D, D), lambda i: (0, 0))
    bspec = pl.BlockSpec((1, D), lambda i: (0, 0))
    rspec = pl.BlockSpec((BR, D), lambda i: (i, 0))
    return pl.pallas_call(
        _proj_body,
        out_shape=[jax.ShapeDtypeStruct((N, D), jnp.float32)] * 3,
        grid=(N // BR,),
        in_specs=[rspec, wspec, bspec, wspec, bspec, wspec, bspec],
        out_specs=[rspec, rspec, rspec],
    )(x, wq, bq.reshape(1, D), wka, bka.reshape(1, D), wma, bma.reshape(1, D))


def _combine_body(p_ref, den_ref, x_ref, wa_ref, ba_ref, beta_ref, o_ref):
    p = p_ref[...]
    dens = den_ref[...]
    agg = p[:, 0] + p[:, 1]
    den = dens[0] + dens[1]
    parts = []
    for h in range(H):
        d = den[:, h:h + 1] + 1e-16
        parts.append(agg[h] / d)
    o = jnp.concatenate(parts, axis=1)
    o = jax.nn.gelu(o)
    o = jnp.dot(o, wa_ref[...], preferred_element_type=jnp.float32) + ba_ref[...]
    beta = beta_ref[0, 0]
    o_ref[...] = beta * o + (1.0 - beta) * x_ref[...]


def _combine(partials, den, x, wa, ba, beta):
    BR = 400
    return pl.pallas_call(
        _combine_body,
        out_shape=jax.ShapeDtypeStruct((N, D), jnp.float32),
        grid=(N // BR,),
        in_specs=[pl.BlockSpec((H, 2, BR, DH), lambda i: (0, 0, i, 0)),
                  pl.BlockSpec((2, BR, 16), lambda i: (0, i, 0)),
                  pl.BlockSpec((BR, D), lambda i: (i, 0)),
                  pl.BlockSpec((D, D), lambda i: (0, 0)),
                  pl.BlockSpec((1, D), lambda i: (0, 0)),
                  pl.BlockSpec((1, 1), lambda i: (0, 0))],
        out_specs=pl.BlockSpec((BR, D), lambda i: (i, 0)),
    )(partials, den, x, wa, ba.reshape(1, D), beta.reshape(1, 1))


def _passa_body(q_hbm, ka_hbm, es_hbm, ed_hbm, w_hbm, den_hbm,
                es0, ed0, es1, ed1, qb0, kab0, qb1, kab1, obw, wout,
                zb, fb, den_sh, si0, si1, sg0, sg1):
    cid = lax.axis_index("c")
    sid = lax.axis_index("s")
    wid = sid * NC + cid
    iota = lax.iota(jnp.int32, 16)
    zeros16 = jnp.zeros((16,), jnp.float32)
    esb = (es0, es1)
    edb = (ed0, ed1)
    qbb = (qb0, qb1)
    kbb = (kab0, kab1)
    sib = (si0, si1)
    sgb = (sg0, sg1)

    def _gbase(c):
        return wid * E_PER_W + c * CH

    def issue_idx(s, c):
        pltpu.async_copy(es_hbm.at[pl.ds(_gbase(c), CH)], esb[s], sib[s])
        pltpu.async_copy(ed_hbm.at[pl.ds(_gbase(c), CH)], edb[s], sib[s])

    def wait_idx(s, c):
        pltpu.make_async_copy(es_hbm.at[pl.ds(_gbase(c), CH)], esb[s], sib[s]).wait()
        pltpu.make_async_copy(ed_hbm.at[pl.ds(_gbase(c), CH)], edb[s], sib[s]).wait()

    def issue_gather(s):
        pltpu.async_copy(q_hbm.at[edb[s]], qbb[s], sgb[s])
        pltpu.async_copy(ka_hbm.at[esb[s]], kbb[s], sgb[s])

    def wait_gather(s):
        pltpu.make_async_copy(q_hbm.at[edb[s]], qbb[s], sgb[s]).wait()
        pltpu.make_async_copy(ka_hbm.at[esb[s]], kbb[s], sgb[s]).wait()

    def _zrow(r, _):
        zb[r, pl.ds(0, 16)] = zeros16
        return 0
    lax.fori_loop(0, DEN_PIECE, _zrow, 0)

    def _orow(r, _):
        obw[r, pl.ds(0, 16)] = zeros16
        return 0
    lax.fori_loop(0, CH, _orow, 0)

    for k in range(K_DEN):
        piece = sid + NS * k
        @pl.when(piece < NP_DEN)
        def _():
            pltpu.sync_copy(zb, den_sh.at[pl.ds(piece * DEN_PIECE, DEN_PIECE)])
    plsc.subcore_barrier()

    issue_idx(0, 0)
    wait_idx(0, 0)
    issue_gather(0)
    issue_idx(1, 1)

    def compute(s, c):
        qb = qbb[s]
        kab = kbb[s]

        def _grp(g, _):
            rows = iota + g * 16
            gid = _gbase(c) + g * 16 + iota
            for h in range(H):
                acc = jnp.zeros((16,), jnp.float32)
                rel = iota
                for d in range(DH):
                    dcol = rel + h * DH
                    acc = acc + (plsc.load_gather(qb, [rows, dcol])
                                 * plsc.load_gather(kab, [rows, dcol]))
                    rel = jnp.bitwise_and(rel + 1, DH - 1)
                w = jnp.exp(acc)
                w = jnp.where(gid < E, w, 0.0)
                hcol = jnp.full((16,), h, jnp.int32)
                plsc.store_scatter(obw, [rows, hcol], w)
                plsc.store_scatter(wout, [rows, hcol], w)
            return 0
        lax.fori_loop(0, CH // 16, _grp, 0)
        pltpu.sync_copy(wout, w_hbm.at[pl.ds(_gbase(c), CH)])
        pltpu.sync_copy(obw, den_sh.at[edb[s]], add=True)

    def _iter(c2, _):
        for par in (0, 1):
            c = 2 * c2 + par
            o = 1 - par

            @pl.when(c + 1 < NCHUNK)
            def _():
                wait_idx(o, c + 1)
                issue_gather(o)
            wait_gather(par)
            compute(par, c)

            @pl.when(c + 2 < NCHUNK)
            def _():
                issue_idx(par, c + 2)
        return 0
    lax.fori_loop(0, NCHUNK // 2, _iter, 0)
    plsc.subcore_barrier()

    for k in range(K_DEN):
        piece = sid + NS * k
        @pl.when(piece < NP_DEN)
        def _():
            start = piece * DEN_PIECE
            pltpu.sync_copy(den_sh.at[pl.ds(start, DEN_PIECE)], fb)
            pltpu.sync_copy(fb, den_hbm.at[cid, pl.ds(start, DEN_PIECE)])


_passa_call = functools.partial(
    pl.kernel,
    _passa_body,
    out_type=[jax.ShapeDtypeStruct((EPAD, 4), jnp.float32),
              jax.ShapeDtypeStruct((NC, N, 16), jnp.float32)],
    mesh=_sc_mesh,
    compiler_params=_sc_params,
    scratch_types=[
        pltpu.VMEM((CH,), jnp.int32),
        pltpu.VMEM((CH,), jnp.int32),
        pltpu.VMEM((CH,), jnp.int32),
        pltpu.VMEM((CH,), jnp.int32),
        pltpu.VMEM((CH, D), jnp.float32),
        pltpu.VMEM((CH, D), jnp.float32),
        pltpu.VMEM((CH, D), jnp.float32),
        pltpu.VMEM((CH, D), jnp.float32),
        pltpu.VMEM((CH, 16), jnp.float32),
        pltpu.VMEM((CH, 4), jnp.float32),
        pltpu.VMEM((DEN_PIECE, 16), jnp.float32),
        pltpu.VMEM((DEN_PIECE, 16), jnp.float32),
        pltpu.VMEM_SHARED((N, 16), jnp.float32),
        pltpu.SemaphoreType.DMA,
        pltpu.SemaphoreType.DMA,
        pltpu.SemaphoreType.DMA,
        pltpu.SemaphoreType.DMA,
    ],
)()


def _passb_body(ma_hbm, w_hbm, es_hbm, ed_hbm, out_hbm,
                es0, ed0, esg0, wb0, mab0, es1, ed1, esg1, wb1, mab1,
                ob, zb, fb, agg_sh, si0, si1, sg0, sg1):
    cid = lax.axis_index("c")
    sid = lax.axis_index("s")
    wid = sid * NC + cid
    iota = lax.iota(jnp.int32, 16)
    zeros16 = jnp.zeros((16,), jnp.float32)
    esb = (es0, es1)
    edb = (ed0, ed1)
    egb = (esg0, esg1)
    wbb = (wb0, wb1)
    mbb = (mab0, mab1)
    sib = (si0, si1)
    sgb = (sg0, sg1)

    def _gbase(c):
        return wid * E_PER_W + c * CH

    def issue_idx(s, c):
        pltpu.async_copy(es_hbm.at[pl.ds(_gbase(c), CH)], esb[s], sib[s])
        pltpu.async_copy(ed_hbm.at[pl.ds(_gbase(c), CH)], edb[s], sib[s])
        pltpu.async_copy(w_hbm.at[pl.ds(_gbase(c), CH)], wbb[s], sib[s])

    def wait_idx(s, c):
        pltpu.make_async_copy(es_hbm.at[pl.ds(_gbase(c), CH)], esb[s], sib[s]).wait()
        pltpu.make_async_copy(ed_hbm.at[pl.ds(_gbase(c), CH)], edb[s], sib[s]).wait()
        pltpu.make_async_copy(w_hbm.at[pl.ds(_gbase(c), CH)], wbb[s], sib[s]).wait()

    def _zrow(r, _):
        for c0 in (0, 16):
            zb[r, pl.ds(c0, 16)] = zeros16
        return 0
    lax.fori_loop(0, AGG_PIECE, _zrow, 0)

    for h in range(H):
        def prep_gather(s, h=h):
            def _gidx(g, _):
                ev = esb[s][pl.ds(g * 16, 16)]
                egb[s][pl.ds(g * 16, 16)] = ev * H + h
                return 0
            lax.fori_loop(0, CH // 16, _gidx, 0)
            pltpu.async_copy(ma_hbm.at[egb[s]], mbb[s], sgb[s])

        def wait_gather(s):
            pltpu.make_async_copy(ma_hbm.at[egb[s]], mbb[s], sgb[s]).wait()

        for k in range(K_AGG):
            piece = sid + NS * k
            @pl.when(piece < NP_AGG)
            def _():
                pltpu.sync_copy(zb, agg_sh.at[pl.ds(piece * AGG_PIECE, AGG_PIECE)])
        plsc.subcore_barrier()

        issue_idx(0, 0)
        wait_idx(0, 0)
        prep_gather(0)
        issue_idx(1, 1)

        def compute(s, h=h):
            mab = mbb[s]

            def _grp(g, _):
                rows = iota + g * 16
                hcol = jnp.full((16,), h, jnp.int32)
                w = plsc.load_gather(wbb[s], [rows, hcol])
                rel = iota
                for d in range(DH):
                    mv = plsc.load_gather(mab, [rows, rel])
                    plsc.store_scatter(ob, [rows, rel], mv * w)
                    rel = jnp.bitwise_and(rel + 1, DH - 1)
                return 0
            lax.fori_loop(0, CH // 16, _grp, 0)
            pltpu.sync_copy(ob, agg_sh.at[edb[s]], add=True)

        def _iter(c2, _):
            for par in (0, 1):
                c = 2 * c2 + par
                o = 1 - par

                @pl.when(c + 1 < NCHUNK)
                def _():
                    wait_idx(o, c + 1)
                    prep_gather(o)
                wait_gather(par)
                compute(par)

                @pl.when(c + 2 < NCHUNK)
                def _():
                    issue_idx(par, c + 2)
            return 0
        lax.fori_loop(0, NCHUNK // 2, _iter, 0)
        plsc.subcore_barrier()

        for k in range(K_AGG):
            piece = sid + NS * k
            @pl.when(piece < NP_AGG)
            def _():
                start = piece * AGG_PIECE
                pltpu.sync_copy(agg_sh.at[pl.ds(start, AGG_PIECE)], fb)
                pltpu.sync_copy(fb, out_hbm.at[h, cid, pl.ds(start, AGG_PIECE)])
        plsc.subcore_barrier()


_passb_call = functools.partial(
    pl.kernel,
    _passb_body,
    out_type=jax.ShapeDtypeStruct((H, NC, N, DH), jnp.float32),
    mesh=_sc_mesh,
    compiler_params=_sc_params,
    scratch_types=[
        pltpu.VMEM((CH,), jnp.int32),
        pltpu.VMEM((CH,), jnp.int32),
        pltpu.VMEM((CH,), jnp.int32),
        pltpu.VMEM((CH, 4), jnp.float32),
        pltpu.VMEM((CH, DH), jnp.float32),
        pltpu.VMEM((CH,), jnp.int32),
        pltpu.VMEM((CH,), jnp.int32),
        pltpu.VMEM((CH,), jnp.int32),
        pltpu.VMEM((CH, 4), jnp.float32),
        pltpu.VMEM((CH, DH), jnp.float32),
        pltpu.VMEM((CH, DH), jnp.float32),
        pltpu.VMEM((AGG_PIECE, DH), jnp.float32),
        pltpu.VMEM((AGG_PIECE, DH), jnp.float32),
        pltpu.VMEM_SHARED((N, DH), jnp.float32),
        pltpu.SemaphoreType.DMA,
        pltpu.SemaphoreType.DMA,
        pltpu.SemaphoreType.DMA,
        pltpu.SemaphoreType.DMA,
    ],
)()


def _score_body(xe_hbm, xn_hbm, i0_hbm, i1_hbm, out_hbm,
                ia0, ib0, ia1, ib1, feb0, fnb0, feb1, fnb1, outb,
                si0, si1, sg0, sg1):
    cid = lax.axis_index("c")
    sid = lax.axis_index("s")
    wid = sid * NC + cid
    iota = lax.iota(jnp.int32, 16)
    iab = (ia0, ia1)
    ibb = (ib0, ib1)
    feb = (feb0, feb1)
    fnb = (fnb0, fnb1)
    sib = (si0, si1)
    sgb = (sg0, sg1)

    def _gbase(c):
        return wid * ELP_PER_W + c * CH

    def issue_idx(s, c):
        pltpu.async_copy(i0_hbm.at[pl.ds(_gbase(c), CH)], iab[s], sib[s])
        pltpu.async_copy(i1_hbm.at[pl.ds(_gbase(c), CH)], ibb[s], sib[s])

    def wait_idx(s, c):
        pltpu.make_async_copy(i0_hbm.at[pl.ds(_gbase(c), CH)], iab[s], sib[s]).wait()
        pltpu.make_async_copy(i1_hbm.at[pl.ds(_gbase(c), CH)], ibb[s], sib[s]).wait()

    def issue_gather(s):
        pltpu.async_copy(xe_hbm.at[iab[s]], feb[s], sgb[s])
        pltpu.async_copy(xn_hbm.at[ibb[s]], fnb[s], sgb[s])

    def wait_gather(s):
        pltpu.make_async_copy(xe_hbm.at[iab[s]], feb[s], sgb[s]).wait()
        pltpu.make_async_copy(xn_hbm.at[ibb[s]], fnb[s], sgb[s]).wait()

    issue_idx(0, 0)
    wait_idx(0, 0)
    issue_gather(0)
    issue_idx(1, 1)

    def compute(s, c):
        def _grp(g, _):
            rows = iota + g * 16
            acc = jnp.zeros((16,), jnp.float32)
            rel = iota
            for d in range(D):
                acc = acc + (plsc.load_gather(feb[s], [rows, rel])
                             * plsc.load_gather(fnb[s], [rows, rel]))
                rel = jnp.bitwise_and(rel + 1, D - 1)
            outb[pl.ds(g * 16, 16)] = acc
            return 0
        lax.fori_loop(0, CH // 16, _grp, 0)
        pltpu.sync_copy(outb, out_hbm.at[pl.ds(_gbase(c), CH)])

    def _iter(c2, _):
        for par in (0, 1):
            c = 2 * c2 + par
            o = 1 - par

            @pl.when(c + 1 < NCHUNK_S)
            def _():
                wait_idx(o, c + 1)
                issue_gather(o)
            wait_gather(par)
            compute(par, c)

            @pl.when(c + 2 < NCHUNK_S)
            def _():
                issue_idx(par, c + 2)
        return 0
    lax.fori_loop(0, NCHUNK_S // 2, _iter, 0)


_score_call = functools.partial(
    pl.kernel,
    _score_body,
    out_type=jax.ShapeDtypeStruct((ELP,), jnp.float32),
    mesh=_sc_mesh,
    compiler_params=_sc_params,
    scratch_types=[
        pltpu.VMEM((CH,), jnp.int32),
        pltpu.VMEM((CH,), jnp.int32),
        pltpu.VMEM((CH,), jnp.int32),
        pltpu.VMEM((CH,), jnp.int32),
        pltpu.VMEM((CH, D), jnp.float32),
        pltpu.VMEM((CH, D), jnp.float32),
        pltpu.VMEM((CH, D), jnp.float32),
        pltpu.VMEM((CH, D), jnp.float32),
        pltpu.VMEM((CH,), jnp.float32),
        pltpu.SemaphoreType.DMA,
        pltpu.SemaphoreType.DMA,
        pltpu.SemaphoreType.DMA,
        pltpu.SemaphoreType.DMA,
    ],
)()


RELS = [('email', 'noun', 'e2n'), ('noun', 'email', 'n2e')]


def kernel(x_email, x_noun, params, edge_index_e2n, edge_index_n2e, edge_label_index):
    p = params
    ei = {'e2n': edge_index_e2n, 'n2e': edge_index_n2e}
    es_pad, ed_pad = {}, {}
    for r in ('e2n', 'n2e'):
        es_pad[r] = jnp.pad(ei[r][0], (0, EPAD - E))
        ed_pad[r] = jnp.pad(ei[r][1], (0, EPAD - E))

    x = {
        'email': _inproj(x_email, p['in_email_W'], p['in_email_b']),
        'noun': _inproj(x_noun, p['in_noun_W'], p['in_noun_b']),
    }

    inv_sqrt_dh = 1.0 / math.sqrt(float(DH))
    for l in range(L):
        folded = {}
        for src, dst, r in RELS:
            arel_s = p['l%d_arel_%s' % (l, r)] * (
                p['l%d_mu_%s' % (l, r)] * inv_sqrt_dh)[:, None, None]
            wka, bka = _fold(p['l%d_K_%s_W' % (l, src)],
                             p['l%d_K_%s_b' % (l, src)], arel_s)
            wma, bma = _fold(p['l%d_V_%s_W' % (l, src)],
                             p['l%d_V_%s_b' % (l, src)],
                             p['l%d_mrel_%s' % (l, r)])
            folded[src] = (wka, bka, wma, bma)

        tabs = {}
        for t in ('email', 'noun'):
            wka, bka, wma, bma = folded[t]
            q, ka, ma = _proj(x[t],
                              p['l%d_Q_%s_W' % (l, t)], p['l%d_Q_%s_b' % (l, t)],
                              wka, bka.reshape(D), wma, bma.reshape(D))
            tabs[t] = (q, ka, ma.reshape(N * H, DH))

        new_x = {}
        for src, dst, r in RELS:
            q_dst = tabs[dst][0]
            ka_src = tabs[src][1]
            ma_src = tabs[src][2]
            w_e, den = _passa_call(q_dst, ka_src, es_pad[r], ed_pad[r])
            partials = _passb_call(ma_src, w_e, es_pad[r], ed_pad[r])
            beta = jax.nn.sigmoid(p['l%d_skip_%s' % (l, dst)])
            new_x[dst] = _combine(partials, den, x[dst],
                                  p['l%d_A_%s_W' % (l, dst)],
                                  p['l%d_A_%s_b' % (l, dst)], beta)
        x = new_x

    eli0 = jnp.pad(edge_label_index[0], (0, ELP - EL))
    eli1 = jnp.pad(edge_label_index[1], (0, ELP - EL))
    out = _score_call(x['email'], x['noun'], eli0, eli1)
    return out[:EL]


# speedup vs baseline: 31.9173x; 1.2707x over previous
import functools
import math

import jax
import jax.numpy as jnp
from jax import lax
from jax.experimental import pallas as pl
from jax.experimental.pallas import tpu as pltpu
from jax.experimental.pallas import tpu_sc as plsc

N = 50000
D = 128
H = 4
DH = D // H
L = 2
E = 400000
EL = 100000

NC = 2
NS = 16
NW = NC * NS

CH = 128
E_PER_W = 25088
EPAD = E_PER_W * NS
NCHUNK = E_PER_W // CH

DEN_PIECE = 200
AGG_PIECE = 200
NP_DEN = N // DEN_PIECE
NP_AGG = N // AGG_PIECE
K_DEN = (NP_DEN + NS - 1) // NS
K_AGG = (NP_AGG + NS - 1) // NS

ELP_PER_W = 3328
ELP = ELP_PER_W * NW
NCHUNK_S = ELP_PER_W // CH

_sc_mesh = plsc.VectorSubcoreMesh(core_axis_name="c", subcore_axis_name="s")

_sc_params = pltpu.CompilerParams(
    needs_layout_passes=False, use_tc_tiling_on_sc=False)


def _fold_body(w_ref, b_ref, rel_ref, wo_ref, bo_ref):
    for h in range(H):
        r = rel_ref[h]
        sl = slice(h * DH, (h + 1) * DH)
        wo_ref[:, sl] = jnp.dot(w_ref[:, sl], r, preferred_element_type=jnp.float32)
        bo_ref[:, sl] = jnp.dot(b_ref[:, sl], r, preferred_element_type=jnp.float32)


def _fold(w, b, rel):
    return pl.pallas_call(
        _fold_body,
        out_shape=[jax.ShapeDtypeStruct((D, D), jnp.float32),
                   jax.ShapeDtypeStruct((1, D), jnp.float32)],
    )(w, b.reshape(1, D), rel)


def _inproj_body(x_ref, w_ref, b_ref, o_ref):
    o_ref[...] = jax.nn.relu(
        jnp.dot(x_ref[...], w_ref[...], preferred_element_type=jnp.float32)
        + b_ref[...])


def _inproj(x, w, b):
    BR = 400
    return pl.pallas_call(
        _inproj_body,
        out_shape=jax.ShapeDtypeStruct((N, D), jnp.float32),
        grid=(N // BR,),
        in_specs=[pl.BlockSpec((BR, D), lambda i: (i, 0)),
                  pl.BlockSpec((D, D), lambda i: (0, 0)),
                  pl.BlockSpec((1, D), lambda i: (0, 0))],
        out_specs=pl.BlockSpec((BR, D), lambda i: (i, 0)),
    )(x, w, b.reshape(1, D))


def _proj_body(x_ref, wq_ref, bq_ref, wka_ref, bka_ref, wma_ref, bma_ref,
               q_ref, ka_ref, ma_ref):
    x = x_ref[...]
    q_ref[...] = jnp.dot(x, wq_ref[...], preferred_element_type=jnp.float32) + bq_ref[...]
    ka_ref[...] = jnp.dot(x, wka_ref[...], preferred_element_type=jnp.float32) + bka_ref[...]
    ma_ref[...] = jnp.dot(x, wma_ref[...], preferred_element_type=jnp.float32) + bma_ref[...]


def _proj(x, wq, bq, wka, bka, wma, bma):
    BR = 400
    wspec = pl.BlockSpec((D, D), lambda i: (0, 0))
    bspec = pl.BlockSpec((1, D), lambda i: (0, 0))
    rspec = pl.BlockSpec((BR, D), lambda i: (i, 0))
    return pl.pallas_call(
        _proj_body,
        out_shape=[jax.ShapeDtypeStruct((N, D), jnp.float32)] * 3,
        grid=(N // BR,),
        in_specs=[rspec, wspec, bspec, wspec, bspec, wspec, bspec],
        out_specs=[rspec, rspec, rspec],
    )(x, wq, bq.reshape(1, D), wka, bka.reshape(1, D), wma, bma.reshape(1, D))


def _combine_body(p_ref, den_ref, x_ref, wa_ref, ba_ref, beta_ref, o_ref):
    p = p_ref[0]
    den = den_ref[0]
    parts = []
    for h in range(H):
        d = den[:, h:h + 1] + 1e-16
        parts.append(p[h] / d)
    o = jnp.concatenate(parts, axis=1)
    o = jax.nn.gelu(o)
    o = jnp.dot(o, wa_ref[...], preferred_element_type=jnp.float32) + ba_ref[...]
    beta = beta_ref[0, 0]
    o_ref[...] = beta * o + (1.0 - beta) * x_ref[...]


def _combine(partials, den, rel, x, wa, ba, beta):
    BR = 400
    return pl.pallas_call(
        _combine_body,
        out_shape=jax.ShapeDtypeStruct((N, D), jnp.float32),
        grid=(N // BR,),
        in_specs=[pl.BlockSpec((1, H, BR, DH), lambda i, rel=rel: (rel, 0, i, 0)),
                  pl.BlockSpec((1, BR, 16), lambda i, rel=rel: (rel, i, 0)),
                  pl.BlockSpec((BR, D), lambda i: (i, 0)),
                  pl.BlockSpec((D, D), lambda i: (0, 0)),
                  pl.BlockSpec((1, D), lambda i: (0, 0)),
                  pl.BlockSpec((1, 1), lambda i: (0, 0))],
        out_specs=pl.BlockSpec((BR, D), lambda i: (i, 0)),
    )(partials, den, x, wa, ba.reshape(1, D), beta.reshape(1, 1))


def _passa_body(q_hbm, ka_hbm, es_hbm, ed_hbm, w_hbm, den_hbm,
                es0, ed0, eq0, es1, ed1, eq1, qb0, kab0, qb1, kab1,
                obw, wout, zb, fb, den_sh, si0, si1, sg0, sg1):
    cid = lax.axis_index("c")
    sid = lax.axis_index("s")
    iota = lax.iota(jnp.int32, 16)
    zeros16 = jnp.zeros((16,), jnp.float32)
    nodeoff = cid * N
    esb = (es0, es1)
    edb = (ed0, ed1)
    eqb = (eq0, eq1)
    qbb = (qb0, qb1)
    kbb = (kab0, kab1)
    sib = (si0, si1)
    sgb = (sg0, sg1)

    def _gbase(c):
        return cid * EPAD + sid * E_PER_W + c * CH

    def issue_idx(s, c):
        pltpu.async_copy(es_hbm.at[pl.ds(_gbase(c), CH)], esb[s], sib[s])
        pltpu.async_copy(ed_hbm.at[pl.ds(_gbase(c), CH)], edb[s], sib[s])

    def wait_idx(s, c):
        pltpu.make_async_copy(es_hbm.at[pl.ds(_gbase(c), CH)], esb[s], sib[s]).wait()
        pltpu.make_async_copy(ed_hbm.at[pl.ds(_gbase(c), CH)], edb[s], sib[s]).wait()

    def prep_gather(s):
        def _gidx(g, _):
            dv = edb[s][pl.ds(g * 16, 16)]
            ev = esb[s][pl.ds(g * 16, 16)]
            eqb[s][pl.ds(g * 16, 16)] = dv + nodeoff
            esb[s][pl.ds(g * 16, 16)] = ev + nodeoff
            return 0
        lax.fori_loop(0, CH // 16, _gidx, 0)
        pltpu.async_copy(q_hbm.at[eqb[s]], qbb[s], sgb[s])
        pltpu.async_copy(ka_hbm.at[esb[s]], kbb[s], sgb[s])

    def wait_gather(s):
        pltpu.make_async_copy(q_hbm.at[eqb[s]], qbb[s], sgb[s]).wait()
        pltpu.make_async_copy(ka_hbm.at[esb[s]], kbb[s], sgb[s]).wait()

    def _zrow(r, _):
        zb[r, pl.ds(0, 16)] = zeros16
        return 0
    lax.fori_loop(0, DEN_PIECE, _zrow, 0)

    def _orow(r, _):
        obw[r, pl.ds(0, 16)] = zeros16
        return 0
    lax.fori_loop(0, CH, _orow, 0)

    for k in range(K_DEN):
        piece = sid + NS * k
        @pl.when(piece < NP_DEN)
        def _():
            pltpu.sync_copy(zb, den_sh.at[pl.ds(piece * DEN_PIECE, DEN_PIECE)])
    plsc.subcore_barrier()

    issue_idx(0, 0)
    wait_idx(0, 0)
    prep_gather(0)
    issue_idx(1, 1)

    def compute(s, c):
        qb = qbb[s]
        kab = kbb[s]

        def _grp(g, _):
            rows = iota + g * 16
            gid = sid * E_PER_W + c * CH + g * 16 + iota
            for h in range(H):
                acc = jnp.zeros((16,), jnp.float32)
                rel = iota
                for d in range(DH):
                    dcol = rel + h * DH
                    acc = acc + (plsc.load_gather(qb, [rows, dcol])
                                 * plsc.load_gather(kab, [rows, dcol]))
                    rel = jnp.bitwise_and(rel + 1, DH - 1)
                w = jnp.exp(acc)
                w = jnp.where(gid < E, w, 0.0)
                hcol = jnp.full((16,), h, jnp.int32)
                plsc.store_scatter(obw, [rows, hcol], w)
                plsc.store_scatter(wout, [rows, hcol], w)
            return 0
        lax.fori_loop(0, CH // 16, _grp, 0)
        pltpu.sync_copy(wout, w_hbm.at[pl.ds(_gbase(c), CH)])
        pltpu.sync_copy(obw, den_sh.at[edb[s]], add=True)

    def _iter(c2, _):
        for par in (0, 1):
            c = 2 * c2 + par
            o = 1 - par

            @pl.when(c + 1 < NCHUNK)
            def _():
                wait_idx(o, c + 1)
                prep_gather(o)
            wait_gather(par)
            compute(par, c)

            @pl.when(c + 2 < NCHUNK)
            def _():
                issue_idx(par, c + 2)
        return 0
    lax.fori_loop(0, NCHUNK // 2, _iter, 0)
    plsc.subcore_barrier()

    for k in range(K_DEN):
        piece = sid + NS * k
        @pl.when(piece < NP_DEN)
        def _():
            start = piece * DEN_PIECE
            pltpu.sync_copy(den_sh.at[pl.ds(start, DEN_PIECE)], fb)
            pltpu.sync_copy(fb, den_hbm.at[cid, pl.ds(start, DEN_PIECE)])


_passa_call = functools.partial(
    pl.kernel,
    _passa_body,
    out_type=[jax.ShapeDtypeStruct((NC * EPAD, 4), jnp.float32),
              jax.ShapeDtypeStruct((NC, N, 16), jnp.float32)],
    mesh=_sc_mesh,
    compiler_params=_sc_params,
    scratch_types=[
        pltpu.VMEM((CH,), jnp.int32),
        pltpu.VMEM((CH,), jnp.int32),
        pltpu.VMEM((CH,), jnp.int32),
        pltpu.VMEM((CH,), jnp.int32),
        pltpu.VMEM((CH,), jnp.int32),
        pltpu.VMEM((CH,), jnp.int32),
        pltpu.VMEM((CH, D), jnp.float32),
        pltpu.VMEM((CH, D), jnp.float32),
        pltpu.VMEM((CH, D), jnp.float32),
        pltpu.VMEM((CH, D), jnp.float32),
        pltpu.VMEM((CH, 16), jnp.float32),
        pltpu.VMEM((CH, 4), jnp.float32),
        pltpu.VMEM((DEN_PIECE, 16), jnp.float32),
        pltpu.VMEM((DEN_PIECE, 16), jnp.float32),
        pltpu.VMEM_SHARED((N, 16), jnp.float32),
        pltpu.SemaphoreType.DMA,
        pltpu.SemaphoreType.DMA,
        pltpu.SemaphoreType.DMA,
        pltpu.SemaphoreType.DMA,
    ],
)()


def _passb_body(ma_hbm, w_hbm, es_hbm, ed_hbm, out_hbm,
                es0, ed0, esg0, wb0, mab0, es1, ed1, esg1, wb1, mab1,
                ob, zb, fb, agg_sh, si0, si1, sg0, sg1):
    cid = lax.axis_index("c")
    sid = lax.axis_index("s")
    iota = lax.iota(jnp.int32, 16)
    zeros16 = jnp.zeros((16,), jnp.float32)
    rowoff = cid * (N * H)
    esb = (es0, es1)
    edb = (ed0, ed1)
    egb = (esg0, esg1)
    wbb = (wb0, wb1)
    mbb = (mab0, mab1)
    sib = (si0, si1)
    sgb = (sg0, sg1)

    def _gbase(c):
        return cid * EPAD + sid * E_PER_W + c * CH

    def issue_idx(s, c):
        pltpu.async_copy(es_hbm.at[pl.ds(_gbase(c), CH)], esb[s], sib[s])
        pltpu.async_copy(ed_hbm.at[pl.ds(_gbase(c), CH)], edb[s], sib[s])
        pltpu.async_copy(w_hbm.at[pl.ds(_gbase(c), CH)], wbb[s], sib[s])

    def wait_idx(s, c):
        pltpu.make_async_copy(es_hbm.at[pl.ds(_gbase(c), CH)], esb[s], sib[s]).wait()
        pltpu.make_async_copy(ed_hbm.at[pl.ds(_gbase(c), CH)], edb[s], sib[s]).wait()
        pltpu.make_async_copy(w_hbm.at[pl.ds(_gbase(c), CH)], wbb[s], sib[s]).wait()

    def _zrow(r, _):
        for c0 in (0, 16):
            zb[r, pl.ds(c0, 16)] = zeros16
        return 0
    lax.fori_loop(0, AGG_PIECE, _zrow, 0)

    for h in range(H):
        def prep_gather(s, h=h):
            def _gidx(g, _):
                ev = esb[s][pl.ds(g * 16, 16)]
                egb[s][pl.ds(g * 16, 16)] = ev * H + h + rowoff
                return 0
            lax.fori_loop(0, CH // 16, _gidx, 0)
            pltpu.async_copy(ma_hbm.at[egb[s]], mbb[s], sgb[s])

        def wait_gather(s):
            pltpu.make_async_copy(ma_hbm.at[egb[s]], mbb[s], sgb[s]).wait()

        for k in range(K_AGG):
            piece = sid + NS * k
            @pl.when(piece < NP_AGG)
            def _():
                pltpu.sync_copy(zb, agg_sh.at[pl.ds(piece * AGG_PIECE, AGG_PIECE)])
        plsc.subcore_barrier()

        issue_idx(0, 0)
        wait_idx(0, 0)
        prep_gather(0)
        issue_idx(1, 1)

        def compute(s, h=h):
            mab = mbb[s]

            def _grp(g, _):
                rows = iota + g * 16
                hcol = jnp.full((16,), h, jnp.int32)
                w = plsc.load_gather(wbb[s], [rows, hcol])
                rel = iota
                for d in range(DH):
                    mv = plsc.load_gather(mab, [rows, rel])
                    plsc.store_scatter(ob, [rows, rel], mv * w)
                    rel = jnp.bitwise_and(rel + 1, DH - 1)
                return 0
            lax.fori_loop(0, CH // 16, _grp, 0)
            pltpu.sync_copy(ob, agg_sh.at[edb[s]], add=True)

        def _iter(c2, _):
            for par in (0, 1):
                c = 2 * c2 + par
                o = 1 - par

                @pl.when(c + 1 < NCHUNK)
                def _():
                    wait_idx(o, c + 1)
                    prep_gather(o)
                wait_gather(par)
                compute(par)

                @pl.when(c + 2 < NCHUNK)
                def _():
                    issue_idx(par, c + 2)
            return 0
        lax.fori_loop(0, NCHUNK // 2, _iter, 0)
        plsc.subcore_barrier()

        for k in range(K_AGG):
            piece = sid + NS * k
            @pl.when(piece < NP_AGG)
            def _():
                start = piece * AGG_PIECE
                pltpu.sync_copy(agg_sh.at[pl.ds(start, AGG_PIECE)], fb)
                pltpu.sync_copy(fb, out_hbm.at[cid, h, pl.ds(start, AGG_PIECE)])
        plsc.subcore_barrier()


_passb_call = functools.partial(
    pl.kernel,
    _passb_body,
    out_type=jax.ShapeDtypeStruct((NC, H, N, DH), jnp.float32),
    mesh=_sc_mesh,
    compiler_params=_sc_params,
    scratch_types=[
        pltpu.VMEM((CH,), jnp.int32),
        pltpu.VMEM((CH,), jnp.int32),
        pltpu.VMEM((CH,), jnp.int32),
        pltpu.VMEM((CH, 4), jnp.float32),
        pltpu.VMEM((CH, DH), jnp.float32),
        pltpu.VMEM((CH,), jnp.int32),
        pltpu.VMEM((CH,), jnp.int32),
        pltpu.VMEM((CH,), jnp.int32),
        pltpu.VMEM((CH, 4), jnp.float32),
        pltpu.VMEM((CH, DH), jnp.float32),
        pltpu.VMEM((CH, DH), jnp.float32),
        pltpu.VMEM((AGG_PIECE, DH), jnp.float32),
        pltpu.VMEM((AGG_PIECE, DH), jnp.float32),
        pltpu.VMEM_SHARED((N, DH), jnp.float32),
        pltpu.SemaphoreType.DMA,
        pltpu.SemaphoreType.DMA,
        pltpu.SemaphoreType.DMA,
        pltpu.SemaphoreType.DMA,
    ],
)()


def _score_body(xe_hbm, xn_hbm, i0_hbm, i1_hbm, out_hbm,
                ia0, ib0, ia1, ib1, feb0, fnb0, feb1, fnb1, outb,
                si0, si1, sg0, sg1):
    cid = lax.axis_index("c")
    sid = lax.axis_index("s")
    wid = sid * NC + cid
    iota = lax.iota(jnp.int32, 16)
    iab = (ia0, ia1)
    ibb = (ib0, ib1)
    feb = (feb0, feb1)
    fnb = (fnb0, fnb1)
    sib = (si0, si1)
    sgb = (sg0, sg1)

    def _gbase(c):
        return wid * ELP_PER_W + c * CH

    def issue_idx(s, c):
        pltpu.async_copy(i0_hbm.at[pl.ds(_gbase(c), CH)], iab[s], sib[s])
        pltpu.async_copy(i1_hbm.at[pl.ds(_gbase(c), CH)], ibb[s], sib[s])

    def wait_idx(s, c):
        pltpu.make_async_copy(i0_hbm.at[pl.ds(_gbase(c), CH)], iab[s], sib[s]).wait()
        pltpu.make_async_copy(i1_hbm.at[pl.ds(_gbase(c), CH)], ibb[s], sib[s]).wait()

    def issue_gather(s):
        pltpu.async_copy(xe_hbm.at[iab[s]], feb[s], sgb[s])
        pltpu.async_copy(xn_hbm.at[ibb[s]], fnb[s], sgb[s])

    def wait_gather(s):
        pltpu.make_async_copy(xe_hbm.at[iab[s]], feb[s], sgb[s]).wait()
        pltpu.make_async_copy(xn_hbm.at[ibb[s]], fnb[s], sgb[s]).wait()

    issue_idx(0, 0)
    wait_idx(0, 0)
    issue_gather(0)
    issue_idx(1, 1)

    def compute(s, c):
        def _grp(g, _):
            rows = iota + g * 16
            acc = jnp.zeros((16,), jnp.float32)
            rel = iota
            for d in range(D):
                acc = acc + (plsc.load_gather(feb[s], [rows, rel])
                             * plsc.load_gather(fnb[s], [rows, rel]))
                rel = jnp.bitwise_and(rel + 1, D - 1)
            outb[pl.ds(g * 16, 16)] = acc
            return 0
        lax.fori_loop(0, CH // 16, _grp, 0)
        pltpu.sync_copy(outb, out_hbm.at[pl.ds(_gbase(c), CH)])

    def _iter(c2, _):
        for par in (0, 1):
            c = 2 * c2 + par
            o = 1 - par

            @pl.when(c + 1 < NCHUNK_S)
            def _():
                wait_idx(o, c + 1)
                issue_gather(o)
            wait_gather(par)
            compute(par, c)

            @pl.when(c + 2 < NCHUNK_S)
            def _():
                issue_idx(par, c + 2)
        return 0
    lax.fori_loop(0, NCHUNK_S // 2, _iter, 0)


_score_call = functools.partial(
    pl.kernel,
    _score_body,
    out_type=jax.ShapeDtypeStruct((ELP,), jnp.float32),
    mesh=_sc_mesh,
    compiler_params=_sc_params,
    scratch_types=[
        pltpu.VMEM((CH,), jnp.int32),
        pltpu.VMEM((CH,), jnp.int32),
        pltpu.VMEM((CH,), jnp.int32),
        pltpu.VMEM((CH,), jnp.int32),
        pltpu.VMEM((CH, D), jnp.float32),
        pltpu.VMEM((CH, D), jnp.float32),
        pltpu.VMEM((CH, D), jnp.float32),
        pltpu.VMEM((CH, D), jnp.float32),
        pltpu.VMEM((CH,), jnp.float32),
        pltpu.SemaphoreType.DMA,
        pltpu.SemaphoreType.DMA,
        pltpu.SemaphoreType.DMA,
        pltpu.SemaphoreType.DMA,
    ],
)()


RELS = [('email', 'noun', 'e2n'), ('noun', 'email', 'n2e')]


def kernel(x_email, x_noun, params, edge_index_e2n, edge_index_n2e, edge_label_index):
    p = params
    ei = {'e2n': edge_index_e2n, 'n2e': edge_index_n2e}
    es2 = jnp.concatenate([jnp.pad(ei[r][0], (0, EPAD - E)) for r in ('e2n', 'n2e')])
    ed2 = jnp.concatenate([jnp.pad(ei[r][1], (0, EPAD - E)) for r in ('e2n', 'n2e')])

    x = {
        'email': _inproj(x_email, p['in_email_W'], p['in_email_b']),
        'noun': _inproj(x_noun, p['in_noun_W'], p['in_noun_b']),
    }

    inv_sqrt_dh = 1.0 / math.sqrt(float(DH))
    for l in range(L):
        folded = {}
        for src, dst, r in RELS:
            arel_s = p['l%d_arel_%s' % (l, r)] * (
                p['l%d_mu_%s' % (l, r)] * inv_sqrt_dh)[:, None, None]
            wka, bka = _fold(p['l%d_K_%s_W' % (l, src)],
                             p['l%d_K_%s_b' % (l, src)], arel_s)
            wma, bma = _fold(p['l%d_V_%s_W' % (l, src)],
                             p['l%d_V_%s_b' % (l, src)],
                             p['l%d_mrel_%s' % (l, r)])
            folded[src] = (wka, bka, wma, bma)

        tabs = {}
        for t in ('email', 'noun'):
            wka, bka, wma, bma = folded[t]
            q, ka, ma = _proj(x[t],
                              p['l%d_Q_%s_W' % (l, t)], p['l%d_Q_%s_b' % (l, t)],
                              wka, bka.reshape(D), wma, bma.reshape(D))
            tabs[t] = (q, ka, ma)

        q_st = jnp.concatenate([tabs['noun'][0], tabs['email'][0]])
        ka_st = jnp.concatenate([tabs['email'][1], tabs['noun'][1]])
        ma_st = jnp.concatenate([tabs['email'][2], tabs['noun'][2]]).reshape(NC * N * H, DH)

        w_e, den = _passa_call(q_st, ka_st, es2, ed2)
        partials = _passb_call(ma_st, w_e, es2, ed2)

        new_x = {}
        for ri, (src, dst, r) in enumerate(RELS):
            beta = jax.nn.sigmoid(p['l%d_skip_%s' % (l, dst)])
            new_x[dst] = _combine(partials, den, ri, x[dst],
                                  p['l%d_A_%s_W' % (l, dst)],
                                  p['l%d_A_%s_b' % (l, dst)], beta)
        x = new_x

    eli0 = jnp.pad(edge_label_index[0], (0, ELP - EL))
    eli1 = jnp.pad(edge_label_index[1], (0, ELP - EL))
    out = _score_call(x['email'], x['noun'], eli0, eli1)
    return out[:EL]


# speedup vs baseline: 32.5877x; 1.0210x over previous
import functools
import math

import jax
import jax.numpy as jnp
from jax import lax
from jax.experimental import pallas as pl
from jax.experimental.pallas import tpu as pltpu
from jax.experimental.pallas import tpu_sc as plsc

N = 50000
D = 128
H = 4
DH = D // H
L = 2
E = 400000
EL = 100000

NC = 2
NS = 16
NW = NC * NS

CH = 128
E_PER_W = 25088
EPAD = E_PER_W * NS
NCHUNK = E_PER_W // CH

DEN_PIECE = 200
AGG_PIECE = 200
NP_DEN = N // DEN_PIECE
NP_AGG = N // AGG_PIECE
K_DEN = (NP_DEN + NS - 1) // NS
K_AGG = (NP_AGG + NS - 1) // NS

ELP_PER_W = 3328
ELP = ELP_PER_W * NW
NCHUNK_S = ELP_PER_W // CH

_sc_mesh = plsc.VectorSubcoreMesh(core_axis_name="c", subcore_axis_name="s")

_sc_params = pltpu.CompilerParams(
    needs_layout_passes=False, use_tc_tiling_on_sc=False)


def _fold_body(w_ref, b_ref, rel_ref, wo_ref, bo_ref):
    for h in range(H):
        r = rel_ref[h]
        sl = slice(h * DH, (h + 1) * DH)
        wo_ref[:, sl] = jnp.dot(w_ref[:, sl], r, preferred_element_type=jnp.float32)
        bo_ref[:, sl] = jnp.dot(b_ref[:, sl], r, preferred_element_type=jnp.float32)


def _fold(w, b, rel):
    return pl.pallas_call(
        _fold_body,
        out_shape=[jax.ShapeDtypeStruct((D, D), jnp.float32),
                   jax.ShapeDtypeStruct((1, D), jnp.float32)],
    )(w, b.reshape(1, D), rel)


BR = 400


def _inproj_body(x_ref, w_ref, b_ref, o_ref):
    o_ref[...] = jax.nn.relu(
        jax.lax.dot_general(x_ref[...], w_ref[...],
                            (((2,), (1,)), ((0,), (0,))),
                            preferred_element_type=jnp.float32)
        + b_ref[...])


def _inproj(x_st, w_st, b_st):
    return pl.pallas_call(
        _inproj_body,
        out_shape=jax.ShapeDtypeStruct((2, N, D), jnp.float32),
        grid=(2, N // BR),
        in_specs=[pl.BlockSpec((1, BR, D), lambda r, i: (r, i, 0)),
                  pl.BlockSpec((1, D, D), lambda r, i: (r, 0, 0)),
                  pl.BlockSpec((1, 1, D), lambda r, i: (r, 0, 0))],
        out_specs=pl.BlockSpec((1, BR, D), lambda r, i: (r, i, 0)),
    )(x_st, w_st, b_st)


def _bdot(x, w):
    return jax.lax.dot_general(x, w, (((2,), (1,)), ((0,), (0,))),
                               preferred_element_type=jnp.float32)


def _proj_body(xd_ref, xs_ref, wq_ref, bq_ref, wka_ref, bka_ref,
               wma_ref, bma_ref, q_ref, ka_ref, ma_ref):
    q_ref[...] = _bdot(xd_ref[...], wq_ref[...]) + bq_ref[...]
    ka_ref[...] = _bdot(xs_ref[...], wka_ref[...]) + bka_ref[...]
    ma_ref[...] = _bdot(xs_ref[...], wma_ref[...]) + bma_ref[...]


def _proj(x_st, wq_st, bq_st, wka_st, bka_st, wma_st, bma_st):
    wspec = pl.BlockSpec((1, D, D), lambda r, i: (r, 0, 0))
    bspec = pl.BlockSpec((1, 1, D), lambda r, i: (r, 0, 0))
    rspec = pl.BlockSpec((1, BR, D), lambda r, i: (r, i, 0))
    xdspec = pl.BlockSpec((1, BR, D), lambda r, i: (1 - r, i, 0))
    return pl.pallas_call(
        _proj_body,
        out_shape=[jax.ShapeDtypeStruct((2, N, D), jnp.float32)] * 3,
        grid=(2, N // BR),
        in_specs=[xdspec, rspec, wspec, bspec, wspec, bspec, wspec, bspec],
        out_specs=[rspec, rspec, rspec],
    )(x_st, x_st, wq_st, bq_st, wka_st, bka_st, wma_st, bma_st)


def _combine_body(p_ref, den_ref, x_ref, wa_ref, ba_ref, beta_ref, o_ref):
    p = p_ref[0]
    den = den_ref[0]
    parts = []
    for h in range(H):
        d = den[:, h:h + 1] + 1e-16
        parts.append(p[h] / d)
    o = jnp.concatenate(parts, axis=1)
    o = jax.nn.gelu(o)
    o = jnp.dot(o, wa_ref[0], preferred_element_type=jnp.float32) + ba_ref[0]
    beta = beta_ref[0, 0, 0]
    o_ref[0] = beta * o + (1.0 - beta) * x_ref[0]


def _combine(partials, den, x_st, wa_st, ba_st, beta_st):
    xflip = pl.BlockSpec((1, BR, D), lambda r, i: (1 - r, i, 0))
    return pl.pallas_call(
        _combine_body,
        out_shape=jax.ShapeDtypeStruct((2, N, D), jnp.float32),
        grid=(2, N // BR),
        in_specs=[pl.BlockSpec((1, H, BR, DH), lambda r, i: (r, 0, i, 0)),
                  pl.BlockSpec((1, BR, 16), lambda r, i: (r, i, 0)),
                  xflip,
                  pl.BlockSpec((1, D, D), lambda r, i: (r, 0, 0)),
                  pl.BlockSpec((1, 1, D), lambda r, i: (r, 0, 0)),
                  pl.BlockSpec((1, 1, 1), lambda r, i: (r, 0, 0))],
        out_specs=xflip,
    )(partials, den, x_st, wa_st, ba_st, beta_st)


def _passa_body(q_hbm, ka_hbm, es_hbm, ed_hbm, w_hbm, den_hbm,
                es0, ed0, eq0, es1, ed1, eq1, qb0, kab0, qb1, kab1,
                obw, wout, zb, fb, den_sh, si0, si1, sg0, sg1):
    cid = lax.axis_index("c")
    sid = lax.axis_index("s")
    iota = lax.iota(jnp.int32, 16)
    zeros16 = jnp.zeros((16,), jnp.float32)
    nodeoff = cid * N
    esb = (es0, es1)
    edb = (ed0, ed1)
    eqb = (eq0, eq1)
    qbb = (qb0, qb1)
    kbb = (kab0, kab1)
    sib = (si0, si1)
    sgb = (sg0, sg1)

    def _gbase(c):
        return cid * EPAD + sid * E_PER_W + c * CH

    def issue_idx(s, c):
        pltpu.async_copy(es_hbm.at[pl.ds(_gbase(c), CH)], esb[s], sib[s])
        pltpu.async_copy(ed_hbm.at[pl.ds(_gbase(c), CH)], edb[s], sib[s])

    def wait_idx(s, c):
        pltpu.make_async_copy(es_hbm.at[pl.ds(_gbase(c), CH)], esb[s], sib[s]).wait()
        pltpu.make_async_copy(ed_hbm.at[pl.ds(_gbase(c), CH)], edb[s], sib[s]).wait()

    def prep_gather(s):
        def _gidx(g, _):
            dv = edb[s][pl.ds(g * 16, 16)]
            ev = esb[s][pl.ds(g * 16, 16)]
            eqb[s][pl.ds(g * 16, 16)] = dv + nodeoff
            esb[s][pl.ds(g * 16, 16)] = ev + nodeoff
            return 0
        lax.fori_loop(0, CH // 16, _gidx, 0)
        pltpu.async_copy(q_hbm.at[eqb[s]], qbb[s], sgb[s])
        pltpu.async_copy(ka_hbm.at[esb[s]], kbb[s], sgb[s])

    def wait_gather(s):
        pltpu.make_async_copy(q_hbm.at[eqb[s]], qbb[s], sgb[s]).wait()
        pltpu.make_async_copy(ka_hbm.at[esb[s]], kbb[s], sgb[s]).wait()

    def _zrow(r, _):
        zb[r, pl.ds(0, 16)] = zeros16
        return 0
    lax.fori_loop(0, DEN_PIECE, _zrow, 0)

    def _orow(r, _):
        obw[r, pl.ds(0, 16)] = zeros16
        return 0
    lax.fori_loop(0, CH, _orow, 0)

    for k in range(K_DEN):
        piece = sid + NS * k
        @pl.when(piece < NP_DEN)
        def _():
            pltpu.sync_copy(zb, den_sh.at[pl.ds(piece * DEN_PIECE, DEN_PIECE)])
    plsc.subcore_barrier()

    issue_idx(0, 0)
    wait_idx(0, 0)
    prep_gather(0)
    issue_idx(1, 1)

    def compute(s, c):
        qb = qbb[s]
        kab = kbb[s]

        def _grp(g, _):
            rows = iota + g * 16
            gid = sid * E_PER_W + c * CH + g * 16 + iota
            for h in range(H):
                acc = jnp.zeros((16,), jnp.float32)
                rel = iota
                for d in range(DH):
                    dcol = rel + h * DH
                    acc = acc + (plsc.load_gather(qb, [rows, dcol])
                                 * plsc.load_gather(kab, [rows, dcol]))
                    rel = jnp.bitwise_and(rel + 1, DH - 1)
                w = jnp.exp(acc)
                w = jnp.where(gid < E, w, 0.0)
                hcol = jnp.full((16,), h, jnp.int32)
                plsc.store_scatter(obw, [rows, hcol], w)
                plsc.store_scatter(wout, [rows, hcol], w)
            return 0
        lax.fori_loop(0, CH // 16, _grp, 0)
        pltpu.sync_copy(wout, w_hbm.at[pl.ds(_gbase(c), CH)])
        pltpu.sync_copy(obw, den_sh.at[edb[s]], add=True)

    def _iter(c2, _):
        for par in (0, 1):
            c = 2 * c2 + par
            o = 1 - par

            @pl.when(c + 1 < NCHUNK)
            def _():
                wait_idx(o, c + 1)
                prep_gather(o)
            wait_gather(par)
            compute(par, c)

            @pl.when(c + 2 < NCHUNK)
            def _():
                issue_idx(par, c + 2)
        return 0
    lax.fori_loop(0, NCHUNK // 2, _iter, 0)
    plsc.subcore_barrier()

    for k in range(K_DEN):
        piece = sid + NS * k
        @pl.when(piece < NP_DEN)
        def _():
            start = piece * DEN_PIECE
            pltpu.sync_copy(den_sh.at[pl.ds(start, DEN_PIECE)], fb)
            pltpu.sync_copy(fb, den_hbm.at[cid, pl.ds(start, DEN_PIECE)])


_passa_call = functools.partial(
    pl.kernel,
    _passa_body,
    out_type=[jax.ShapeDtypeStruct((NC * EPAD, 4), jnp.float32),
              jax.ShapeDtypeStruct((NC, N, 16), jnp.float32)],
    mesh=_sc_mesh,
    compiler_params=_sc_params,
    scratch_types=[
        pltpu.VMEM((CH,), jnp.int32),
        pltpu.VMEM((CH,), jnp.int32),
        pltpu.VMEM((CH,), jnp.int32),
        pltpu.VMEM((CH,), jnp.int32),
        pltpu.VMEM((CH,), jnp.int32),
        pltpu.VMEM((CH,), jnp.int32),
        pltpu.VMEM((CH, D), jnp.float32),
        pltpu.VMEM((CH, D), jnp.float32),
        pltpu.VMEM((CH, D), jnp.float32),
        pltpu.VMEM((CH, D), jnp.float32),
        pltpu.VMEM((CH, 16), jnp.float32),
        pltpu.VMEM((CH, 4), jnp.float32),
        pltpu.VMEM((DEN_PIECE, 16), jnp.float32),
        pltpu.VMEM((DEN_PIECE, 16), jnp.float32),
        pltpu.VMEM_SHARED((N, 16), jnp.float32),
        pltpu.SemaphoreType.DMA,
        pltpu.SemaphoreType.DMA,
        pltpu.SemaphoreType.DMA,
        pltpu.SemaphoreType.DMA,
    ],
)()


def _passb_body(ma_hbm, w_hbm, es_hbm, ed_hbm, out_hbm,
                es0, ed0, esg0, wb0, mab0, es1, ed1, esg1, wb1, mab1,
                ob, zb, fb, agg_sh, si0, si1, sg0, sg1):
    cid = lax.axis_index("c")
    sid = lax.axis_index("s")
    iota = lax.iota(jnp.int32, 16)
    zeros16 = jnp.zeros((16,), jnp.float32)
    rowoff = cid * (N * H)
    esb = (es0, es1)
    edb = (ed0, ed1)
    egb = (esg0, esg1)
    wbb = (wb0, wb1)
    mbb = (mab0, mab1)
    sib = (si0, si1)
    sgb = (sg0, sg1)

    def _gbase(c):
        return cid * EPAD + sid * E_PER_W + c * CH

    def issue_idx(s, c):
        pltpu.async_copy(es_hbm.at[pl.ds(_gbase(c), CH)], esb[s], sib[s])
        pltpu.async_copy(ed_hbm.at[pl.ds(_gbase(c), CH)], edb[s], sib[s])
        pltpu.async_copy(w_hbm.at[pl.ds(_gbase(c), CH)], wbb[s], sib[s])

    def wait_idx(s, c):
        pltpu.make_async_copy(es_hbm.at[pl.ds(_gbase(c), CH)], esb[s], sib[s]).wait()
        pltpu.make_async_copy(ed_hbm.at[pl.ds(_gbase(c), CH)], edb[s], sib[s]).wait()
        pltpu.make_async_copy(w_hbm.at[pl.ds(_gbase(c), CH)], wbb[s], sib[s]).wait()

    def _zrow(r, _):
        for c0 in (0, 16):
            zb[r, pl.ds(c0, 16)] = zeros16
        return 0
    lax.fori_loop(0, AGG_PIECE, _zrow, 0)

    for h in range(H):
        def prep_gather(s, h=h):
            def _gidx(g, _):
                ev = esb[s][pl.ds(g * 16, 16)]
                egb[s][pl.ds(g * 16, 16)] = ev * H + h + rowoff
                return 0
            lax.fori_loop(0, CH // 16, _gidx, 0)
            pltpu.async_copy(ma_hbm.at[egb[s]], mbb[s], sgb[s])

        def wait_gather(s):
            pltpu.make_async_copy(ma_hbm.at[egb[s]], mbb[s], sgb[s]).wait()

        for k in range(K_AGG):
            piece = sid + NS * k
            @pl.when(piece < NP_AGG)
            def _():
                pltpu.sync_copy(zb, agg_sh.at[pl.ds(piece * AGG_PIECE, AGG_PIECE)])
        plsc.subcore_barrier()

        issue_idx(0, 0)
        wait_idx(0, 0)
        prep_gather(0)
        issue_idx(1, 1)

        def compute(s, h=h):
            mab = mbb[s]

            def _grp(g, _):
                rows = iota + g * 16
                hcol = jnp.full((16,), h, jnp.int32)
                w = plsc.load_gather(wbb[s], [rows, hcol])
                rel = iota
                for d in range(DH):
                    mv = plsc.load_gather(mab, [rows, rel])
                    plsc.store_scatter(ob, [rows, rel], mv * w)
                    rel = jnp.bitwise_and(rel + 1, DH - 1)
                return 0
            lax.fori_loop(0, CH // 16, _grp, 0)
            pltpu.sync_copy(ob, agg_sh.at[edb[s]], add=True)

        def _iter(c2, _):
            for par in (0, 1):
                c = 2 * c2 + par
                o = 1 - par

                @pl.when(c + 1 < NCHUNK)
                def _():
                    wait_idx(o, c + 1)
                    prep_gather(o)
                wait_gather(par)
                compute(par)

                @pl.when(c + 2 < NCHUNK)
                def _():
                    issue_idx(par, c + 2)
            return 0
        lax.fori_loop(0, NCHUNK // 2, _iter, 0)
        plsc.subcore_barrier()

        for k in range(K_AGG):
            piece = sid + NS * k
            @pl.when(piece < NP_AGG)
            def _():
                start = piece * AGG_PIECE
                pltpu.sync_copy(agg_sh.at[pl.ds(start, AGG_PIECE)], fb)
                pltpu.sync_copy(fb, out_hbm.at[cid, h, pl.ds(start, AGG_PIECE)])
        plsc.subcore_barrier()


_passb_call = functools.partial(
    pl.kernel,
    _passb_body,
    out_type=jax.ShapeDtypeStruct((NC, H, N, DH), jnp.float32),
    mesh=_sc_mesh,
    compiler_params=_sc_params,
    scratch_types=[
        pltpu.VMEM((CH,), jnp.int32),
        pltpu.VMEM((CH,), jnp.int32),
        pltpu.VMEM((CH,), jnp.int32),
        pltpu.VMEM((CH, 4), jnp.float32),
        pltpu.VMEM((CH, DH), jnp.float32),
        pltpu.VMEM((CH,), jnp.int32),
        pltpu.VMEM((CH,), jnp.int32),
        pltpu.VMEM((CH,), jnp.int32),
        pltpu.VMEM((CH, 4), jnp.float32),
        pltpu.VMEM((CH, DH), jnp.float32),
        pltpu.VMEM((CH, DH), jnp.float32),
        pltpu.VMEM((AGG_PIECE, DH), jnp.float32),
        pltpu.VMEM((AGG_PIECE, DH), jnp.float32),
        pltpu.VMEM_SHARED((N, DH), jnp.float32),
        pltpu.SemaphoreType.DMA,
        pltpu.SemaphoreType.DMA,
        pltpu.SemaphoreType.DMA,
        pltpu.SemaphoreType.DMA,
    ],
)()


def _score_body(x2_hbm, i0_hbm, i1_hbm, out_hbm,
                ia0, ib0, ia1, ib1, feb0, fnb0, feb1, fnb1, outb,
                si0, si1, sg0, sg1):
    cid = lax.axis_index("c")
    sid = lax.axis_index("s")
    wid = sid * NC + cid
    iota = lax.iota(jnp.int32, 16)
    iab = (ia0, ia1)
    ibb = (ib0, ib1)
    feb = (feb0, feb1)
    fnb = (fnb0, fnb1)
    sib = (si0, si1)
    sgb = (sg0, sg1)

    def _gbase(c):
        return wid * ELP_PER_W + c * CH

    def issue_idx(s, c):
        pltpu.async_copy(i0_hbm.at[pl.ds(_gbase(c), CH)], iab[s], sib[s])
        pltpu.async_copy(i1_hbm.at[pl.ds(_gbase(c), CH)], ibb[s], sib[s])

    def wait_idx(s, c):
        pltpu.make_async_copy(i0_hbm.at[pl.ds(_gbase(c), CH)], iab[s], sib[s]).wait()
        pltpu.make_async_copy(i1_hbm.at[pl.ds(_gbase(c), CH)], ibb[s], sib[s]).wait()

    def issue_gather(s):
        def _adj(g, _):
            v = ibb[s][pl.ds(g * 16, 16)]
            ibb[s][pl.ds(g * 16, 16)] = v + N
            return 0
        lax.fori_loop(0, CH // 16, _adj, 0)
        pltpu.async_copy(x2_hbm.at[iab[s]], feb[s], sgb[s])
        pltpu.async_copy(x2_hbm.at[ibb[s]], fnb[s], sgb[s])

    def wait_gather(s):
        pltpu.make_async_copy(x2_hbm.at[iab[s]], feb[s], sgb[s]).wait()
        pltpu.make_async_copy(x2_hbm.at[ibb[s]], fnb[s], sgb[s]).wait()

    issue_idx(0, 0)
    wait_idx(0, 0)
    issue_gather(0)
    issue_idx(1, 1)

    def compute(s, c):
        def _grp(g, _):
            rows = iota + g * 16
            acc = jnp.zeros((16,), jnp.float32)
            rel = iota
            for d in range(D):
                acc = acc + (plsc.load_gather(feb[s], [rows, rel])
                             * plsc.load_gather(fnb[s], [rows, rel]))
                rel = jnp.bitwise_and(rel + 1, D - 1)
            outb[pl.ds(g * 16, 16)] = acc
            return 0
        lax.fori_loop(0, CH // 16, _grp, 0)
        pltpu.sync_copy(outb, out_hbm.at[pl.ds(_gbase(c), CH)])

    def _iter(c2, _):
        for par in (0, 1):
            c = 2 * c2 + par
            o = 1 - par

            @pl.when(c + 1 < NCHUNK_S)
            def _():
                wait_idx(o, c + 1)
                issue_gather(o)
            wait_gather(par)
            compute(par, c)

            @pl.when(c + 2 < NCHUNK_S)
            def _():
                issue_idx(par, c + 2)
        return 0
    lax.fori_loop(0, NCHUNK_S // 2, _iter, 0)


_score_call = functools.partial(
    pl.kernel,
    _score_body,
    out_type=jax.ShapeDtypeStruct((ELP,), jnp.float32),
    mesh=_sc_mesh,
    compiler_params=_sc_params,
    scratch_types=[
        pltpu.VMEM((CH,), jnp.int32),
        pltpu.VMEM((CH,), jnp.int32),
        pltpu.VMEM((CH,), jnp.int32),
        pltpu.VMEM((CH,), jnp.int32),
        pltpu.VMEM((CH, D), jnp.float32),
        pltpu.VMEM((CH, D), jnp.float32),
        pltpu.VMEM((CH, D), jnp.float32),
        pltpu.VMEM((CH, D), jnp.float32),
        pltpu.VMEM((CH,), jnp.float32),
        pltpu.SemaphoreType.DMA,
        pltpu.SemaphoreType.DMA,
        pltpu.SemaphoreType.DMA,
        pltpu.SemaphoreType.DMA,
    ],
)()


RELS = [('email', 'noun', 'e2n'), ('noun', 'email', 'n2e')]


def kernel(x_email, x_noun, params, edge_index_e2n, edge_index_n2e, edge_label_index):
    p = params
    ei = {'e2n': edge_index_e2n, 'n2e': edge_index_n2e}
    es2 = jnp.concatenate([jnp.pad(ei[r][0], (0, EPAD - E)) for r in ('e2n', 'n2e')])
    ed2 = jnp.concatenate([jnp.pad(ei[r][1], (0, EPAD - E)) for r in ('e2n', 'n2e')])

    x_st = _inproj(
        jnp.stack([x_email, x_noun]),
        jnp.stack([p['in_email_W'], p['in_noun_W']]),
        jnp.stack([p['in_email_b'], p['in_noun_b']]).reshape(2, 1, D))

    inv_sqrt_dh = 1.0 / math.sqrt(float(DH))
    for l in range(L):
        folded = {}
        for src, dst, r in RELS:
            arel_s = p['l%d_arel_%s' % (l, r)] * (
                p['l%d_mu_%s' % (l, r)] * inv_sqrt_dh)[:, None, None]
            wka, bka = _fold(p['l%d_K_%s_W' % (l, src)],
                             p['l%d_K_%s_b' % (l, src)], arel_s)
            wma, bma = _fold(p['l%d_V_%s_W' % (l, src)],
                             p['l%d_V_%s_b' % (l, src)],
                             p['l%d_mrel_%s' % (l, r)])
            folded[src] = (wka, bka, wma, bma)

        wq_st = jnp.stack([p['l%d_Q_noun_W' % l], p['l%d_Q_email_W' % l]])
        bq_st = jnp.stack([p['l%d_Q_noun_b' % l], p['l%d_Q_email_b' % l]]).reshape(2, 1, D)
        wka_st = jnp.stack([folded['email'][0], folded['noun'][0]])
        bka_st = jnp.stack([folded['email'][1], folded['noun'][1]]).reshape(2, 1, D)
        wma_st = jnp.stack([folded['email'][2], folded['noun'][2]])
        bma_st = jnp.stack([folded['email'][3], folded['noun'][3]]).reshape(2, 1, D)

        q3, ka3, ma3 = _proj(x_st, wq_st, bq_st, wka_st, bka_st, wma_st, bma_st)

        w_e, den = _passa_call(q3.reshape(NC * N, D), ka3.reshape(NC * N, D),
                               es2, ed2)
        partials = _passb_call(ma3.reshape(NC * N * H, DH), w_e, es2, ed2)

        wa_st = jnp.stack([p['l%d_A_noun_W' % l], p['l%d_A_email_W' % l]])
        ba_st = jnp.stack([p['l%d_A_noun_b' % l], p['l%d_A_email_b' % l]]).reshape(2, 1, D)
        beta_st = jax.nn.sigmoid(jnp.stack(
            [p['l%d_skip_noun' % l], p['l%d_skip_email' % l]])).reshape(2, 1, 1)
        x_st = _combine(partials, den, x_st, wa_st, ba_st, beta_st)

    eli0 = jnp.pad(edge_label_index[0], (0, ELP - EL))
    eli1 = jnp.pad(edge_label_index[1], (0, ELP - EL))
    out = _score_call(x_st.reshape(NC * N, D), eli0, eli1)
    return out[:EL]
